# Initial kernel scaffold; baseline (speedup 1.0000x reference)
import jax
import jax.numpy as jnp
from jax.experimental import pallas as pl


def kernel(emb, W1, b1, W2, b2, Wr1, root1, bias1, Wr2, root2, bias2, edge_index, edge_type):
    raise NotImplementedError("write your pallas kernel here")


# speedup vs baseline: 1.1812x; 1.1812x over previous
import functools
import jax
import jax.numpy as jnp
from jax.experimental import pallas as pl

N = 50000
E = 800000
R = 8
EMB = 64
HID = 64
LABELS = 16
OUT_F = 112

ROW_BLK = 400


def _dense1_body(emb_ref, w1t_ref, b1_ref, w2t_ref, b2_ref, wr1f_ref,
                 root1_ref, bias1_ref, xw_ref, xr_ref):
    e = emb_ref[...]
    x = jax.nn.sigmoid(
        jnp.dot(e, w1t_ref[...], preferred_element_type=jnp.float32)
        + b1_ref[...])
    x = jax.nn.sigmoid(
        jnp.dot(x, w2t_ref[...], preferred_element_type=jnp.float32)
        + b2_ref[...])
    xw_ref[...] = jnp.dot(x, wr1f_ref[...], preferred_element_type=jnp.float32)
    xr_ref[...] = (
        jnp.dot(x, root1_ref[...], preferred_element_type=jnp.float32)
        + bias1_ref[...])


def _dense1(emb, W1t, b1, W2t, b2, Wr1f, root1, bias1):
    grid = (N // ROW_BLK,)
    return pl.pallas_call(
        _dense1_body,
        grid=grid,
        in_specs=[
            pl.BlockSpec((ROW_BLK, EMB), lambda i: (i, 0)),
            pl.BlockSpec((EMB, OUT_F), lambda i: (0, 0)),
            pl.BlockSpec((1, OUT_F), lambda i: (0, 0)),
            pl.BlockSpec((OUT_F, EMB), lambda i: (0, 0)),
            pl.BlockSpec((1, EMB), lambda i: (0, 0)),
            pl.BlockSpec((EMB, R * HID), lambda i: (0, 0)),
            pl.BlockSpec((EMB, HID), lambda i: (0, 0)),
            pl.BlockSpec((1, HID), lambda i: (0, 0)),
        ],
        out_specs=[
            pl.BlockSpec((ROW_BLK, R * HID), lambda i: (i, 0)),
            pl.BlockSpec((ROW_BLK, HID), lambda i: (i, 0)),
        ],
        out_shape=[
            jax.ShapeDtypeStruct((N, R * HID), jnp.float32),
            jax.ShapeDtypeStruct((N, HID), jnp.float32),
        ],
    )(emb, W1t, b1, W2t, b2, Wr1f, root1, bias1)


def _dense2_body(agg_ref, xr_ref, wr2f_ref, root2_ref, bias2_ref,
                 hw_ref, hr_ref):
    h = jax.nn.relu(agg_ref[...] + xr_ref[...])
    hw_ref[...] = jnp.dot(h, wr2f_ref[...], preferred_element_type=jnp.float32)
    hr_ref[...] = (
        jnp.dot(h, root2_ref[...], preferred_element_type=jnp.float32)
        + bias2_ref[...])


def _dense2(agg1, xr1, Wr2f, root2, bias2):
    grid = (N // ROW_BLK,)
    return pl.pallas_call(
        _dense2_body,
        grid=grid,
        in_specs=[
            pl.BlockSpec((ROW_BLK, HID), lambda i: (i, 0)),
            pl.BlockSpec((ROW_BLK, HID), lambda i: (i, 0)),
            pl.BlockSpec((HID, R * LABELS), lambda i: (0, 0)),
            pl.BlockSpec((HID, LABELS), lambda i: (0, 0)),
            pl.BlockSpec((1, LABELS), lambda i: (0, 0)),
        ],
        out_specs=[
            pl.BlockSpec((ROW_BLK, R * LABELS), lambda i: (i, 0)),
            pl.BlockSpec((ROW_BLK, LABELS), lambda i: (i, 0)),
        ],
        out_shape=[
            jax.ShapeDtypeStruct((N, R * LABELS), jnp.float32),
            jax.ShapeDtypeStruct((N, LABELS), jnp.float32),
        ],
    )(agg1, xr1, Wr2f, root2, bias2)


def _final_body(agg_ref, hr_ref, out_ref):
    out_ref[...] = jax.nn.sigmoid(agg_ref[...] + hr_ref[...])


def _final(agg2, hr2):
    grid = (N // ROW_BLK,)
    return pl.pallas_call(
        _final_body,
        grid=grid,
        in_specs=[
            pl.BlockSpec((ROW_BLK, LABELS), lambda i: (i, 0)),
            pl.BlockSpec((ROW_BLK, LABELS), lambda i: (i, 0)),
        ],
        out_specs=pl.BlockSpec((ROW_BLK, LABELS), lambda i: (i, 0)),
        out_shape=jax.ShapeDtypeStruct((N, LABELS), jnp.float32),
    )(agg2, hr2)


def kernel(emb, W1, b1, W2, b2, Wr1, root1, bias1, Wr2, root2, bias2,
           edge_index, edge_type):
    src = edge_index[0]
    dst = edge_index[1]
    comp = dst * R + edge_type
    flat = src * R + edge_type

    counts = jax.ops.segment_sum(
        jnp.ones((E,), jnp.float32), comp, num_segments=N * R)
    inv = jnp.where(counts > 0, 1.0 / jnp.maximum(counts, 1.0), 0.0)
    edge_scale = inv[comp]

    W1t = W1.T
    W2t = W2.T
    Wr1f = Wr1.transpose(1, 0, 2).reshape(EMB, R * HID)
    Wr2f = Wr2.transpose(1, 0, 2).reshape(HID, R * LABELS)

    xW1, xr1 = _dense1(emb, W1t, b1.reshape(1, -1), W2t, b2.reshape(1, -1),
                       Wr1f, root1, bias1.reshape(1, -1))

    msgs1 = xW1.reshape(N * R, HID)[flat] * edge_scale[:, None]
    agg1 = jax.ops.segment_sum(msgs1, dst, num_segments=N)

    hW2, hr2 = _dense2(agg1, xr1, Wr2f, root2, bias2.reshape(1, -1))

    msgs2 = hW2.reshape(N * R, LABELS)[flat] * edge_scale[:, None]
    agg2 = jax.ops.segment_sum(msgs2, dst, num_segments=N)

    return _final(agg2, hr2)


# speedup vs baseline: 1.7192x; 1.4555x over previous
import functools
import jax
import jax.numpy as jnp
from jax import lax
from jax.experimental import pallas as pl
from jax.experimental.pallas import tpu as pltpu
from jax.experimental.pallas import tpu_sc as plsc

N = 50000
E = 800000
R = 8
EMB = 64
HID = 64
LABELS = 16
OUT_F = 112

ROW_BLK = 400

NSC = 2
NTILE = 16
NR = N * R
NR_PAD = NTILE * 25024
SLICE_W = NR_PAD // NTILE
CHUNK = 3200
CHUNK_ROWS = CHUNK // 128
N_CHUNKS = E // CHUNK

_SC_MESH = plsc.VectorSubcoreMesh(
    core_axis_name="c", subcore_axis_name="s",
    num_cores=NSC, num_subcores=NTILE)


def _counts_body(dst_hbm, typ_hbm, inv_hbm, dst_v, typ_v, comp_v, ones_v,
                 sweep_v, table, sem):
    cid = lax.axis_index("c")
    sid = lax.axis_index("s")

    def _fill(i, _):
        ones_v[pl.ds(i * 16, 16)] = jnp.full((16,), 1.0, jnp.float32)
        return 0
    lax.fori_loop(0, 128 // 16, _fill, 0)

    def _zero(i, _):
        sweep_v[pl.ds(i * 16, 16)] = jnp.zeros((16,), jnp.float32)
        return 0
    lax.fori_loop(0, SLICE_W // 16, _zero, 0)

    pltpu.sync_copy(sweep_v, table.at[pl.ds(sid * SLICE_W, SLICE_W)])
    plsc.subcore_barrier()

    def _chunk(k, _):
        base = (sid + k * NTILE) * CHUNK
        pltpu.sync_copy(dst_hbm.at[pl.ds(base, CHUNK)], dst_v)
        pltpu.sync_copy(typ_hbm.at[pl.ds(base, CHUNK)], typ_v)

        def _comp(j, _):
            def _grp(u, _):
                off = j * 128 + u * 16
                d = dst_v[pl.ds(off, 16)]
                t = typ_v[pl.ds(off, 16)]
                comp_v[j, pl.ds(u * 16, 16)] = d * R + t
                return 0
            lax.fori_loop(0, 8, _grp, 0)
            return 0
        lax.fori_loop(0, CHUNK_ROWS, _comp, 0)

        def _scat(j, _):
            pltpu.sync_copy(ones_v, table.at[comp_v.at[j]], add=True)
            return 0
        lax.fori_loop(0, CHUNK_ROWS, _scat, 0)
        return 0

    n_mine = N_CHUNKS // NTILE + jnp.where(sid < (N_CHUNKS % NTILE), 1, 0)
    lax.fori_loop(0, n_mine, _chunk, 0)
    plsc.subcore_barrier()

    pltpu.sync_copy(table.at[pl.ds(sid * SLICE_W, SLICE_W)], sweep_v)

    def _inv(i, _):
        c = sweep_v[pl.ds(i * 16, 16)]
        sweep_v[pl.ds(i * 16, 16)] = jnp.where(
            c > 0.0, 1.0 / jnp.maximum(c, 1.0), 0.0)
        return 0
    lax.fori_loop(0, SLICE_W // 16, _inv, 0)

    @pl.when((sid < NTILE // 2) == (cid == 0))
    def _():
        pltpu.sync_copy(sweep_v, inv_hbm.at[pl.ds(sid * SLICE_W, SLICE_W)])


@functools.partial(
    pl.kernel,
    out_type=jax.ShapeDtypeStruct((NR_PAD,), jnp.float32),
    mesh=_SC_MESH,
    scratch_types=[
        pltpu.VMEM((CHUNK,), jnp.int32),
        pltpu.VMEM((CHUNK,), jnp.int32),
        pltpu.VMEM((CHUNK_ROWS, 128), jnp.int32),
        pltpu.VMEM((128,), jnp.float32),
        pltpu.VMEM((SLICE_W,), jnp.float32),
        pltpu.VMEM_SHARED((NR_PAD,), jnp.float32),
        pltpu.SemaphoreType.DMA,
    ],
)
def _sc_counts(dst_hbm, typ_hbm, inv_hbm, dst_v, typ_v, comp_v, ones_v,
               sweep_v, table, sem):
    _counts_body(dst_hbm, typ_hbm, inv_hbm, dst_v, typ_v, comp_v, ones_v,
                 sweep_v, table, sem)


def _dense1_body(emb_ref, w1t_ref, b1_ref, w2t_ref, b2_ref, wr1f_ref,
                 root1_ref, bias1_ref, xw_ref, xr_ref):
    e = emb_ref[...]
    x = jax.nn.sigmoid(
        jnp.dot(e, w1t_ref[...], preferred_element_type=jnp.float32)
        + b1_ref[...])
    x = jax.nn.sigmoid(
        jnp.dot(x, w2t_ref[...], preferred_element_type=jnp.float32)
        + b2_ref[...])
    xw_ref[...] = jnp.dot(x, wr1f_ref[...], preferred_element_type=jnp.float32)
    xr_ref[...] = (
        jnp.dot(x, root1_ref[...], preferred_element_type=jnp.float32)
        + bias1_ref[...])


def _dense1(emb, W1t, b1, W2t, b2, Wr1f, root1, bias1):
    grid = (N // ROW_BLK,)
    return pl.pallas_call(
        _dense1_body,
        grid=grid,
        in_specs=[
            pl.BlockSpec((ROW_BLK, EMB), lambda i: (i, 0)),
            pl.BlockSpec((EMB, OUT_F), lambda i: (0, 0)),
            pl.BlockSpec((1, OUT_F), lambda i: (0, 0)),
            pl.BlockSpec((OUT_F, EMB), lambda i: (0, 0)),
            pl.BlockSpec((1, EMB), lambda i: (0, 0)),
            pl.BlockSpec((EMB, R * HID), lambda i: (0, 0)),
            pl.BlockSpec((EMB, HID), lambda i: (0, 0)),
            pl.BlockSpec((1, HID), lambda i: (0, 0)),
        ],
        out_specs=[
            pl.BlockSpec((ROW_BLK, R * HID), lambda i: (i, 0)),
            pl.BlockSpec((ROW_BLK, HID), lambda i: (i, 0)),
        ],
        out_shape=[
            jax.ShapeDtypeStruct((N, R * HID), jnp.float32),
            jax.ShapeDtypeStruct((N, HID), jnp.float32),
        ],
    )(emb, W1t, b1, W2t, b2, Wr1f, root1, bias1)


def _dense2_body(agg_ref, xr_ref, wr2f_ref, root2_ref, bias2_ref,
                 hw_ref, hr_ref):
    h = jax.nn.relu(agg_ref[...] + xr_ref[...])
    hw_ref[...] = jnp.dot(h, wr2f_ref[...], preferred_element_type=jnp.float32)
    hr_ref[...] = (
        jnp.dot(h, root2_ref[...], preferred_element_type=jnp.float32)
        + bias2_ref[...])


def _dense2(agg1, xr1, Wr2f, root2, bias2):
    grid = (N // ROW_BLK,)
    return pl.pallas_call(
        _dense2_body,
        grid=grid,
        in_specs=[
            pl.BlockSpec((ROW_BLK, HID), lambda i: (i, 0)),
            pl.BlockSpec((ROW_BLK, HID), lambda i: (i, 0)),
            pl.BlockSpec((HID, R * LABELS), lambda i: (0, 0)),
            pl.BlockSpec((HID, LABELS), lambda i: (0, 0)),
            pl.BlockSpec((1, LABELS), lambda i: (0, 0)),
        ],
        out_specs=[
            pl.BlockSpec((ROW_BLK, R * LABELS), lambda i: (i, 0)),
            pl.BlockSpec((ROW_BLK, LABELS), lambda i: (i, 0)),
        ],
        out_shape=[
            jax.ShapeDtypeStruct((N, R * LABELS), jnp.float32),
            jax.ShapeDtypeStruct((N, LABELS), jnp.float32),
        ],
    )(agg1, xr1, Wr2f, root2, bias2)


def _final_body(agg_ref, hr_ref, out_ref):
    out_ref[...] = jax.nn.sigmoid(agg_ref[...] + hr_ref[...])


def _final(agg2, hr2):
    grid = (N // ROW_BLK,)
    return pl.pallas_call(
        _final_body,
        grid=grid,
        in_specs=[
            pl.BlockSpec((ROW_BLK, LABELS), lambda i: (i, 0)),
            pl.BlockSpec((ROW_BLK, LABELS), lambda i: (i, 0)),
        ],
        out_specs=pl.BlockSpec((ROW_BLK, LABELS), lambda i: (i, 0)),
        out_shape=jax.ShapeDtypeStruct((N, LABELS), jnp.float32),
    )(agg2, hr2)


def kernel(emb, W1, b1, W2, b2, Wr1, root1, bias1, Wr2, root2, bias2,
           edge_index, edge_type):
    src = edge_index[0]
    dst = edge_index[1]
    comp = dst * R + edge_type
    flat = src * R + edge_type

    inv = _sc_counts(dst, edge_type)
    edge_scale = inv[comp]

    W1t = W1.T
    W2t = W2.T
    Wr1f = Wr1.transpose(1, 0, 2).reshape(EMB, R * HID)
    Wr2f = Wr2.transpose(1, 0, 2).reshape(HID, R * LABELS)

    xW1, xr1 = _dense1(emb, W1t, b1.reshape(1, -1), W2t, b2.reshape(1, -1),
                       Wr1f, root1, bias1.reshape(1, -1))

    msgs1 = xW1.reshape(N * R, HID)[flat] * edge_scale[:, None]
    agg1 = jax.ops.segment_sum(msgs1, dst, num_segments=N)

    hW2, hr2 = _dense2(agg1, xr1, Wr2f, root2, bias2.reshape(1, -1))

    msgs2 = hW2.reshape(N * R, LABELS)[flat] * edge_scale[:, None]
    agg2 = jax.ops.segment_sum(msgs2, dst, num_segments=N)

    return _final(agg2, hr2)


# speedup vs baseline: 14.0351x; 8.1638x over previous
import functools
import jax
import jax.numpy as jnp
from jax import lax
from jax.experimental import pallas as pl
from jax.experimental.pallas import tpu as pltpu
from jax.experimental.pallas import tpu_sc as plsc

N = 50000
E = 800000
R = 8
EMB = 64
HID = 64
LABELS = 16
OUT_F = 112

ROW_BLK = 400

NSC = 2
NTILE = 16
NR = N * R
SLICE_W = 25024
NR_PAD = NTILE * SLICE_W
NROW = 3128
NROW_LAST = N - 15 * NROW

_SC_MESH = plsc.VectorSubcoreMesh(
    core_axis_name="c", subcore_axis_name="s",
    num_cores=NSC, num_subcores=NTILE)
_SC_PARAMS = pltpu.CompilerParams(use_tc_tiling_on_sc=False)


C_CNT = 3200
CNT_ROWS = C_CNT // 128
CNT_CHUNKS = E // C_CNT
SCL_CHUNKS = E // NSC // C_CNT


def _counts_body(dst_hbm, typ_hbm, scl_hbm, dst_v, typ_v, comp_v, ones_v,
                 sweep_v, invv, table, sem):
    cid = lax.axis_index("c")
    sid = lax.axis_index("s")

    def _fill(i, _):
        ones_v[pl.ds(i * 16, 16)] = jnp.full((16,), 1.0, jnp.float32)
        return 0
    lax.fori_loop(0, 128 // 16, _fill, 0)

    def _zero(i, _):
        sweep_v[pl.ds(i * 16, 16)] = jnp.zeros((16,), jnp.float32)
        return 0
    lax.fori_loop(0, SLICE_W // 16, _zero, 0)

    pltpu.sync_copy(sweep_v, table.at[pl.ds(sid * SLICE_W, SLICE_W)])
    plsc.subcore_barrier()

    def _chunk(k, _):
        base = (sid + k * NTILE) * C_CNT
        pltpu.sync_copy(dst_hbm.at[pl.ds(base, C_CNT)], dst_v)
        pltpu.sync_copy(typ_hbm.at[pl.ds(base, C_CNT)], typ_v)

        def _comp(j, _):
            def _grp(u, _):
                off = j * 128 + u * 16
                d = dst_v[pl.ds(off, 16)]
                t = typ_v[pl.ds(off, 16)]
                comp_v[j, pl.ds(u * 16, 16)] = d * R + t
                return 0
            lax.fori_loop(0, 8, _grp, 0)
            return 0
        lax.fori_loop(0, CNT_ROWS, _comp, 0)

        def _scat(j, _):
            pltpu.sync_copy(ones_v, table.at[comp_v.at[j]], add=True)
            return 0
        lax.fori_loop(0, CNT_ROWS, _scat, 0)
        return 0

    n_mine = CNT_CHUNKS // NTILE + jnp.where(sid < (CNT_CHUNKS % NTILE), 1, 0)
    lax.fori_loop(0, n_mine, _chunk, 0)
    plsc.subcore_barrier()

    pltpu.sync_copy(table.at[pl.ds(sid * SLICE_W, SLICE_W)], sweep_v)

    def _inv(i, _):
        c = sweep_v[pl.ds(i * 16, 16)]
        sweep_v[pl.ds(i * 16, 16)] = jnp.where(
            c > 0.0, 1.0 / jnp.maximum(c, 1.0), 0.0)
        return 0
    lax.fori_loop(0, SLICE_W // 16, _inv, 0)
    pltpu.sync_copy(sweep_v, table.at[pl.ds(sid * SLICE_W, SLICE_W)])
    plsc.subcore_barrier()

    def _schunk(k, _):
        base = cid * (E // NSC) + (sid + k * NTILE) * C_CNT
        pltpu.sync_copy(dst_hbm.at[pl.ds(base, C_CNT)], dst_v)
        pltpu.sync_copy(typ_hbm.at[pl.ds(base, C_CNT)], typ_v)

        def _comp(j, _):
            def _grp(u, _):
                off = j * 128 + u * 16
                d = dst_v[pl.ds(off, 16)]
                t = typ_v[pl.ds(off, 16)]
                comp_v[j, pl.ds(u * 16, 16)] = d * R + t
                return 0
            lax.fori_loop(0, 8, _grp, 0)
            return 0
        lax.fori_loop(0, CNT_ROWS, _comp, 0)

        def _gat(j, _):
            pltpu.sync_copy(table.at[comp_v.at[j]],
                            invv.at[pl.ds(j * 128, 128)])
            return 0
        lax.fori_loop(0, CNT_ROWS, _gat, 0)
        pltpu.sync_copy(invv, scl_hbm.at[pl.ds(base, C_CNT)])
        return 0

    n_mine2 = SCL_CHUNKS // NTILE + jnp.where(sid < (SCL_CHUNKS % NTILE), 1, 0)
    lax.fori_loop(0, n_mine2, _schunk, 0)


@functools.partial(
    pl.kernel,
    out_type=jax.ShapeDtypeStruct((E,), jnp.float32),
    mesh=_SC_MESH,
    compiler_params=_SC_PARAMS,
    scratch_types=[
        pltpu.VMEM((C_CNT,), jnp.int32),
        pltpu.VMEM((C_CNT,), jnp.int32),
        pltpu.VMEM((CNT_ROWS, 128), jnp.int32),
        pltpu.VMEM((128,), jnp.float32),
        pltpu.VMEM((SLICE_W,), jnp.float32),
        pltpu.VMEM((C_CNT,), jnp.float32),
        pltpu.VMEM_SHARED((NR_PAD,), jnp.float32),
        pltpu.SemaphoreType.DMA,
    ],
)
def _sc_counts(dst_hbm, typ_hbm, scl_hbm, dst_v, typ_v, comp_v, ones_v,
               sweep_v, invv, table, sem):
    _counts_body(dst_hbm, typ_hbm, scl_hbm, dst_v, typ_v, comp_v, ones_v,
                 sweep_v, invv, table, sem)


C1 = 640
C1_ROWS = C1 // 128
C1_CHUNKS = E // C1


def _msg1_body(src_hbm, dst_hbm, typ_hbm, scl_hbm, xw_hbm, xr_hbm, h_hbm,
               src_v, dst_v, typ_v, scl_v, flat2d, dst2d, rows_v, acc, sem):
    cid = lax.axis_index("c")
    sid = lax.axis_index("s")
    tbl_off = cid * NR

    def _init(nrows):
        for off in range(0, 3200, C1):
            ln = min(C1, nrows - off)
            if ln <= 0:
                break
            pltpu.sync_copy(xr_hbm.at[cid, pl.ds(sid * NROW + off, ln)],
                            rows_v.at[pl.ds(0, ln)])
            pltpu.sync_copy(rows_v.at[pl.ds(0, ln)],
                            acc.at[pl.ds(sid * NROW + off, ln)])

    @pl.when(sid < NTILE - 1)
    def _():
        _init(NROW)

    @pl.when(sid == NTILE - 1)
    def _():
        _init(NROW_LAST)
    plsc.subcore_barrier()

    def _chunk(k, _):
        base = (sid + k * NTILE) * C1
        pltpu.sync_copy(src_hbm.at[pl.ds(base, C1)], src_v)
        pltpu.sync_copy(dst_hbm.at[pl.ds(base, C1)], dst_v)
        pltpu.sync_copy(typ_hbm.at[pl.ds(base, C1)], typ_v)
        pltpu.sync_copy(scl_hbm.at[pl.ds(base, C1)], scl_v)

        def _idx(j, _):
            def _grp(u, _):
                off = j * 128 + u * 16
                s = src_v[pl.ds(off, 16)]
                d = dst_v[pl.ds(off, 16)]
                t = typ_v[pl.ds(off, 16)]
                flat2d[j, pl.ds(u * 16, 16)] = s * R + t + tbl_off
                dst2d[j, pl.ds(u * 16, 16)] = d
                return 0
            lax.fori_loop(0, 8, _grp, 0)
            return 0
        lax.fori_loop(0, C1_ROWS, _idx, 0)

        descs = [pltpu.async_copy(xw_hbm.at[flat2d.at[j]],
                                  rows_v.at[pl.ds(j * 128, 128)], sem)
                 for j in range(C1_ROWS)]
        for d_ in descs:
            d_.wait()

        def _scale(g, _):
            sv = scl_v[pl.ds(g * 16, 16)]
            for u in range(16):
                e = g * 16 + u
                s = sv[u]
                rows_v[e, pl.ds(0, 16)] = rows_v[e, pl.ds(0, 16)] * s
                rows_v[e, pl.ds(16, 16)] = rows_v[e, pl.ds(16, 16)] * s
            return 0
        lax.fori_loop(0, C1 // 16, _scale, 0)

        for j in range(C1_ROWS):
            pltpu.sync_copy(rows_v.at[pl.ds(j * 128, 128)],
                            acc.at[dst2d.at[j]], add=True)
        return 0

    n_mine = C1_CHUNKS // NTILE + jnp.where(sid < (C1_CHUNKS % NTILE), 1, 0)
    lax.fori_loop(0, n_mine, _chunk, 0)
    plsc.subcore_barrier()

    def _wout(nrows):
        for off in range(0, 3200, C1):
            ln = min(C1, nrows - off)
            if ln <= 0:
                break
            pltpu.sync_copy(acc.at[pl.ds(sid * NROW + off, ln)],
                            rows_v.at[pl.ds(0, ln)])
            pltpu.sync_copy(rows_v.at[pl.ds(0, ln)],
                            h_hbm.at[cid, pl.ds(sid * NROW + off, ln)])

    @pl.when(sid < NTILE - 1)
    def _():
        _wout(NROW)

    @pl.when(sid == NTILE - 1)
    def _():
        _wout(NROW_LAST)


@functools.partial(
    pl.kernel,
    out_type=jax.ShapeDtypeStruct((NSC, N, HID // 2), jnp.float32),
    mesh=_SC_MESH,
    compiler_params=_SC_PARAMS,
    scratch_types=[
        pltpu.VMEM((C1,), jnp.int32),
        pltpu.VMEM((C1,), jnp.int32),
        pltpu.VMEM((C1,), jnp.int32),
        pltpu.VMEM((C1,), jnp.float32),
        pltpu.VMEM((C1_ROWS, 128), jnp.int32),
        pltpu.VMEM((C1_ROWS, 128), jnp.int32),
        pltpu.VMEM((C1, HID // 2), jnp.float32),
        pltpu.VMEM_SHARED((N, HID // 2), jnp.float32),
        pltpu.SemaphoreType.DMA,
    ],
)
def _sc_msg1(src_hbm, dst_hbm, typ_hbm, scl_hbm, xw_hbm, xr_hbm, h_hbm,
             src_v, dst_v, typ_v, scl_v, flat2d, dst2d, rows_v, acc, sem):
    _msg1_body(src_hbm, dst_hbm, typ_hbm, scl_hbm, xw_hbm, xr_hbm, h_hbm,
               src_v, dst_v, typ_v, scl_v, flat2d, dst2d, rows_v, acc, sem)


C2 = 3200
C2_ROWS = C2 // 128
C2_CHUNKS = E // NSC // C2


def _msg2_body(src_hbm, dst_hbm, typ_hbm, scl_hbm, hw_hbm, hr_hbm, p_hbm,
               src_v, dst_v, typ_v, scl_v, flat2d, dst2d, rows_v, acc, sem):
    cid = lax.axis_index("c")
    sid = lax.axis_index("s")

    @pl.when(cid == 0)
    def _():
        @pl.when(sid < NTILE - 1)
        def _():
            pltpu.sync_copy(hr_hbm.at[pl.ds(sid * NROW, NROW)],
                            rows_v.at[pl.ds(0, NROW)])
            pltpu.sync_copy(rows_v.at[pl.ds(0, NROW)],
                            acc.at[pl.ds(sid * NROW, NROW)])

        @pl.when(sid == NTILE - 1)
        def _():
            pltpu.sync_copy(hr_hbm.at[pl.ds(15 * NROW, NROW_LAST)],
                            rows_v.at[pl.ds(0, NROW_LAST)])
            pltpu.sync_copy(rows_v.at[pl.ds(0, NROW_LAST)],
                            acc.at[pl.ds(15 * NROW, NROW_LAST)])

    @pl.when(cid == 1)
    def _():
        def _z(i, _):
            rows_v[i, pl.ds(0, 16)] = jnp.zeros((16,), jnp.float32)
            return 0
        lax.fori_loop(0, NROW, _z, 0)

        @pl.when(sid < NTILE - 1)
        def _():
            pltpu.sync_copy(rows_v.at[pl.ds(0, NROW)],
                            acc.at[pl.ds(sid * NROW, NROW)])

        @pl.when(sid == NTILE - 1)
        def _():
            pltpu.sync_copy(rows_v.at[pl.ds(0, NROW_LAST)],
                            acc.at[pl.ds(15 * NROW, NROW_LAST)])
    plsc.subcore_barrier()

    def _chunk(k, _):
        base = (cid * (E // NSC)) + (sid + k * NTILE) * C2
        pltpu.sync_copy(src_hbm.at[pl.ds(base, C2)], src_v)
        pltpu.sync_copy(dst_hbm.at[pl.ds(base, C2)], dst_v)
        pltpu.sync_copy(typ_hbm.at[pl.ds(base, C2)], typ_v)
        pltpu.sync_copy(scl_hbm.at[pl.ds(base, C2)], scl_v)

        def _idx(j, _):
            def _grp(u, _):
                off = j * 128 + u * 16
                s = src_v[pl.ds(off, 16)]
                d = dst_v[pl.ds(off, 16)]
                t = typ_v[pl.ds(off, 16)]
                flat2d[j, pl.ds(u * 16, 16)] = s * R + t
                dst2d[j, pl.ds(u * 16, 16)] = d
                return 0
            lax.fori_loop(0, 8, _grp, 0)
            return 0
        lax.fori_loop(0, C2_ROWS, _idx, 0)

        descs = [pltpu.async_copy(hw_hbm.at[flat2d.at[j]],
                                  rows_v.at[pl.ds(j * 128, 128)], sem)
                 for j in range(C2_ROWS)]
        for d_ in descs:
            d_.wait()

        def _scale(g, _):
            sv = scl_v[pl.ds(g * 16, 16)]
            for u in range(16):
                e = g * 16 + u
                rows_v[e, pl.ds(0, 16)] = rows_v[e, pl.ds(0, 16)] * sv[u]
            return 0
        lax.fori_loop(0, C2 // 16, _scale, 0)

        for j in range(C2_ROWS):
            pltpu.sync_copy(rows_v.at[pl.ds(j * 128, 128)],
                            acc.at[dst2d.at[j]], add=True)
        return 0

    n_mine = C2_CHUNKS // NTILE + jnp.where(sid < (C2_CHUNKS % NTILE), 1, 0)
    lax.fori_loop(0, n_mine, _chunk, 0)
    plsc.subcore_barrier()

    @pl.when(sid < NTILE - 1)
    def _():
        pltpu.sync_copy(acc.at[pl.ds(sid * NROW, NROW)],
                        rows_v.at[pl.ds(0, NROW)])
        pltpu.sync_copy(rows_v.at[pl.ds(0, NROW)],
                        p_hbm.at[cid, pl.ds(sid * NROW, NROW)])

    @pl.when(sid == NTILE - 1)
    def _():
        pltpu.sync_copy(acc.at[pl.ds(15 * NROW, NROW_LAST)],
                        rows_v.at[pl.ds(0, NROW_LAST)])
        pltpu.sync_copy(rows_v.at[pl.ds(0, NROW_LAST)],
                        p_hbm.at[cid, pl.ds(15 * NROW, NROW_LAST)])


@functools.partial(
    pl.kernel,
    out_type=jax.ShapeDtypeStruct((NSC, N, LABELS), jnp.float32),
    mesh=_SC_MESH,
    compiler_params=_SC_PARAMS,
    scratch_types=[
        pltpu.VMEM((C2,), jnp.int32),
        pltpu.VMEM((C2,), jnp.int32),
        pltpu.VMEM((C2,), jnp.int32),
        pltpu.VMEM((C2,), jnp.float32),
        pltpu.VMEM((C2_ROWS, 128), jnp.int32),
        pltpu.VMEM((C2_ROWS, 128), jnp.int32),
        pltpu.VMEM((C2, LABELS), jnp.float32),
        pltpu.VMEM_SHARED((N, LABELS), jnp.float32),
        pltpu.SemaphoreType.DMA,
    ],
)
def _sc_msg2(src_hbm, dst_hbm, typ_hbm, scl_hbm, hw_hbm, hr_hbm, p_hbm,
             src_v, dst_v, typ_v, scl_v, flat2d, dst2d, rows_v, acc, sem):
    _msg2_body(src_hbm, dst_hbm, typ_hbm, scl_hbm, hw_hbm, hr_hbm, p_hbm,
               src_v, dst_v, typ_v, scl_v, flat2d, dst2d, rows_v, acc, sem)


def _dense1_body(emb_ref, w1t_ref, b1_ref, w2t_ref, b2_ref, wr1a_ref,
                 wr1b_ref, roota_ref, rootb_ref, ba_ref, bb_ref,
                 xw_ref, xr_ref):
    e = emb_ref[...]
    x = jax.nn.sigmoid(
        jnp.dot(e, w1t_ref[...], preferred_element_type=jnp.float32)
        + b1_ref[...])
    x = jax.nn.sigmoid(
        jnp.dot(x, w2t_ref[...], preferred_element_type=jnp.float32)
        + b2_ref[...])
    xw_ref[0] = jnp.dot(x, wr1a_ref[...], preferred_element_type=jnp.float32)
    xw_ref[1] = jnp.dot(x, wr1b_ref[...], preferred_element_type=jnp.float32)
    xr_ref[0] = (jnp.dot(x, roota_ref[...], preferred_element_type=jnp.float32)
                 + ba_ref[...])
    xr_ref[1] = (jnp.dot(x, rootb_ref[...], preferred_element_type=jnp.float32)
                 + bb_ref[...])


def _dense1(emb, W1t, b1, W2t, b2, Wr1a, Wr1b, roota, rootb, ba, bb):
    grid = (N // ROW_BLK,)
    return pl.pallas_call(
        _dense1_body,
        grid=grid,
        in_specs=[
            pl.BlockSpec((ROW_BLK, EMB), lambda i: (i, 0)),
            pl.BlockSpec((EMB, OUT_F), lambda i: (0, 0)),
            pl.BlockSpec((1, OUT_F), lambda i: (0, 0)),
            pl.BlockSpec((OUT_F, EMB), lambda i: (0, 0)),
            pl.BlockSpec((1, EMB), lambda i: (0, 0)),
            pl.BlockSpec((EMB, R * HID // 2), lambda i: (0, 0)),
            pl.BlockSpec((EMB, R * HID // 2), lambda i: (0, 0)),
            pl.BlockSpec((EMB, HID // 2), lambda i: (0, 0)),
            pl.BlockSpec((EMB, HID // 2), lambda i: (0, 0)),
            pl.BlockSpec((1, HID // 2), lambda i: (0, 0)),
            pl.BlockSpec((1, HID // 2), lambda i: (0, 0)),
        ],
        out_specs=[
            pl.BlockSpec((NSC, ROW_BLK, R * HID // 2), lambda i: (0, i, 0)),
            pl.BlockSpec((NSC, ROW_BLK, HID // 2), lambda i: (0, i, 0)),
        ],
        out_shape=[
            jax.ShapeDtypeStruct((NSC, N, R * HID // 2), jnp.float32),
            jax.ShapeDtypeStruct((NSC, N, HID // 2), jnp.float32),
        ],
    )(emb, W1t, b1, W2t, b2, Wr1a, Wr1b, roota, rootb, ba, bb)


def _dense2_body(ha_ref, hb_ref, wr2a_ref, wr2b_ref, root2a_ref, root2b_ref,
                 bias2_ref, hw_ref, hr_ref):
    ha = jax.nn.relu(ha_ref[...])
    hb = jax.nn.relu(hb_ref[...])
    hw_ref[...] = (
        jnp.dot(ha, wr2a_ref[...], preferred_element_type=jnp.float32)
        + jnp.dot(hb, wr2b_ref[...], preferred_element_type=jnp.float32))
    hr_ref[...] = (
        jnp.dot(ha, root2a_ref[...], preferred_element_type=jnp.float32)
        + jnp.dot(hb, root2b_ref[...], preferred_element_type=jnp.float32)
        + bias2_ref[...])


def _dense2(ha, hb, Wr2a, Wr2b, root2a, root2b, bias2):
    grid = (N // ROW_BLK,)
    return pl.pallas_call(
        _dense2_body,
        grid=grid,
        in_specs=[
            pl.BlockSpec((ROW_BLK, HID // 2), lambda i: (i, 0)),
            pl.BlockSpec((ROW_BLK, HID // 2), lambda i: (i, 0)),
            pl.BlockSpec((HID // 2, R * LABELS), lambda i: (0, 0)),
            pl.BlockSpec((HID // 2, R * LABELS), lambda i: (0, 0)),
            pl.BlockSpec((HID // 2, LABELS), lambda i: (0, 0)),
            pl.BlockSpec((HID // 2, LABELS), lambda i: (0, 0)),
            pl.BlockSpec((1, LABELS), lambda i: (0, 0)),
        ],
        out_specs=[
            pl.BlockSpec((ROW_BLK, R * LABELS), lambda i: (i, 0)),
            pl.BlockSpec((ROW_BLK, LABELS), lambda i: (i, 0)),
        ],
        out_shape=[
            jax.ShapeDtypeStruct((N, R * LABELS), jnp.float32),
            jax.ShapeDtypeStruct((N, LABELS), jnp.float32),
        ],
    )(ha, hb, Wr2a, Wr2b, root2a, root2b, bias2)


def _final_body(p_ref, out_ref):
    out_ref[...] = jax.nn.sigmoid(p_ref[0] + p_ref[1])


def _final(p):
    grid = (N // ROW_BLK,)
    return pl.pallas_call(
        _final_body,
        grid=grid,
        in_specs=[pl.BlockSpec((NSC, ROW_BLK, LABELS), lambda i: (0, i, 0))],
        out_specs=pl.BlockSpec((ROW_BLK, LABELS), lambda i: (i, 0)),
        out_shape=jax.ShapeDtypeStruct((N, LABELS), jnp.float32),
    )(p)


def kernel(emb, W1, b1, W2, b2, Wr1, root1, bias1, Wr2, root2, bias2,
           edge_index, edge_type):
    src = edge_index[0]
    dst = edge_index[1]

    edge_scale = _sc_counts(dst, edge_type)

    W1t = W1.T
    W2t = W2.T
    Wr1f = Wr1.transpose(1, 0, 2)
    Wr1a = Wr1f[:, :, :HID // 2].reshape(EMB, R * HID // 2)
    Wr1b = Wr1f[:, :, HID // 2:].reshape(EMB, R * HID // 2)
    roota = root1[:, :HID // 2]
    rootb = root1[:, HID // 2:]

    xw, xr = _dense1(emb, W1t, b1.reshape(1, -1), W2t, b2.reshape(1, -1),
                     Wr1a, Wr1b, roota, rootb,
                     bias1[:HID // 2].reshape(1, -1),
                     bias1[HID // 2:].reshape(1, -1))

    h_raw = _sc_msg1(src, dst, edge_type, edge_scale,
                     xw.reshape(NSC * NR, HID // 2), xr)

    Wr2f = Wr2.transpose(1, 0, 2)
    Wr2a = Wr2f[:HID // 2].reshape(HID // 2, R * LABELS)
    Wr2b = Wr2f[HID // 2:].reshape(HID // 2, R * LABELS)
    root2a = root2[:HID // 2]
    root2b = root2[HID // 2:]

    hw, hr = _dense2(h_raw[0], h_raw[1], Wr2a, Wr2b, root2a, root2b,
                     bias2.reshape(1, -1))

    p = _sc_msg2(src, dst, edge_type, edge_scale, hw.reshape(NR, LABELS), hr)

    return _final(p)


# speedup vs baseline: 18.6544x; 1.3291x over previous
import functools
import jax
import jax.numpy as jnp
from jax import lax
from jax.experimental import pallas as pl
from jax.experimental.pallas import tpu as pltpu
from jax.experimental.pallas import tpu_sc as plsc

N = 50000
E = 800000
R = 8
EMB = 64
HID = 64
LABELS = 16
OUT_F = 112

ROW_BLK = 400

NSC = 2
NTILE = 16
NR = N * R
SLICE_W = 25024
NR_PAD = NTILE * SLICE_W
NROW = 3128
NROW_LAST = N - 15 * NROW

_SC_MESH = plsc.VectorSubcoreMesh(
    core_axis_name="c", subcore_axis_name="s",
    num_cores=NSC, num_subcores=NTILE)
_SC_PARAMS = pltpu.CompilerParams(use_tc_tiling_on_sc=False)


C_CNT = 3200
CNT_ROWS = C_CNT // 128
CNT_CHUNKS = E // C_CNT
SCL_CHUNKS = E // NSC // C_CNT


def _counts_body(dst_hbm, typ_hbm, scl_hbm, dst_v, typ_v, comp_v, ones_v,
                 sweep_v, invv, table, sem, sem2):
    cid = lax.axis_index("c")
    sid = lax.axis_index("s")

    def _fill(i, _):
        ones_v[pl.ds(i * 16, 16)] = jnp.full((16,), 1.0, jnp.float32)
        return 0
    lax.fori_loop(0, 128 // 16, _fill, 0)

    def _zero(i, _):
        sweep_v[pl.ds(i * 16, 16)] = jnp.zeros((16,), jnp.float32)
        return 0
    lax.fori_loop(0, SLICE_W // 16, _zero, 0)

    pltpu.sync_copy(sweep_v, table.at[pl.ds(sid * SLICE_W, SLICE_W)])
    plsc.subcore_barrier()

    def _chunk(k, _):
        base = (sid + k * NTILE) * C_CNT
        stg = [pltpu.async_copy(dst_hbm.at[pl.ds(base, C_CNT)], dst_v, sem),
               pltpu.async_copy(typ_hbm.at[pl.ds(base, C_CNT)], typ_v, sem)]
        for d_ in stg:
            d_.wait()

        def _comp(j, _):
            def _grp(u, _):
                off = j * 128 + u * 16
                d = dst_v[pl.ds(off, 16)]
                t = typ_v[pl.ds(off, 16)]
                comp_v[j, pl.ds(u * 16, 16)] = d * R + t
                return 0
            lax.fori_loop(0, 8, _grp, 0)
            return 0
        lax.fori_loop(0, CNT_ROWS, _comp, 0)

        scds = [pltpu.async_copy(ones_v, table.at[comp_v.at[j]], sem2,
                                 add=True)
                for j in range(CNT_ROWS)]
        for d_ in scds:
            d_.wait()
        return 0

    n_mine = CNT_CHUNKS // NTILE + jnp.where(sid < (CNT_CHUNKS % NTILE), 1, 0)
    lax.fori_loop(0, n_mine, _chunk, 0)
    plsc.subcore_barrier()

    pltpu.sync_copy(table.at[pl.ds(sid * SLICE_W, SLICE_W)], sweep_v)

    def _inv(i, _):
        c = sweep_v[pl.ds(i * 16, 16)]
        sweep_v[pl.ds(i * 16, 16)] = jnp.where(
            c > 0.0, 1.0 / jnp.maximum(c, 1.0), 0.0)
        return 0
    lax.fori_loop(0, SLICE_W // 16, _inv, 0)
    pltpu.sync_copy(sweep_v, table.at[pl.ds(sid * SLICE_W, SLICE_W)])
    plsc.subcore_barrier()

    def _schunk(k, _):
        base = cid * (E // NSC) + (sid + k * NTILE) * C_CNT
        pltpu.sync_copy(dst_hbm.at[pl.ds(base, C_CNT)], dst_v)
        pltpu.sync_copy(typ_hbm.at[pl.ds(base, C_CNT)], typ_v)

        def _comp(j, _):
            def _grp(u, _):
                off = j * 128 + u * 16
                d = dst_v[pl.ds(off, 16)]
                t = typ_v[pl.ds(off, 16)]
                comp_v[j, pl.ds(u * 16, 16)] = d * R + t
                return 0
            lax.fori_loop(0, 8, _grp, 0)
            return 0
        lax.fori_loop(0, CNT_ROWS, _comp, 0)

        gds = [pltpu.async_copy(table.at[comp_v.at[j]],
                                invv.at[pl.ds(j * 128, 128)], sem2)
               for j in range(CNT_ROWS)]
        for d_ in gds:
            d_.wait()
        pltpu.sync_copy(invv, scl_hbm.at[pl.ds(base, C_CNT)])
        return 0

    n_mine2 = SCL_CHUNKS // NTILE + jnp.where(sid < (SCL_CHUNKS % NTILE), 1, 0)
    lax.fori_loop(0, n_mine2, _schunk, 0)


@functools.partial(
    pl.kernel,
    out_type=jax.ShapeDtypeStruct((E,), jnp.float32),
    mesh=_SC_MESH,
    compiler_params=_SC_PARAMS,
    scratch_types=[
        pltpu.VMEM((C_CNT,), jnp.int32),
        pltpu.VMEM((C_CNT,), jnp.int32),
        pltpu.VMEM((CNT_ROWS, 128), jnp.int32),
        pltpu.VMEM((128,), jnp.float32),
        pltpu.VMEM((SLICE_W,), jnp.float32),
        pltpu.VMEM((C_CNT,), jnp.float32),
        pltpu.VMEM_SHARED((NR_PAD,), jnp.float32),
        pltpu.SemaphoreType.DMA,
        pltpu.SemaphoreType.DMA,
    ],
)
def _sc_counts(dst_hbm, typ_hbm, scl_hbm, dst_v, typ_v, comp_v, ones_v,
               sweep_v, invv, table, sem, sem2):
    _counts_body(dst_hbm, typ_hbm, scl_hbm, dst_v, typ_v, comp_v, ones_v,
                 sweep_v, invv, table, sem, sem2)


C1 = 640
C1_ROWS = C1 // 128
C1_CHUNKS = E // C1


def _msg1_body(src_hbm, dst_hbm, typ_hbm, scl_hbm, xw_hbm, xr_hbm, h_hbm,
               src_v, dst_v, typ_v, scl_v, flat2d, dst2d, rows_v, acc, sem,
               sem2):
    cid = lax.axis_index("c")
    sid = lax.axis_index("s")
    tbl_off = cid * NR

    def _init(nrows):
        for off in range(0, 3200, C1):
            ln = min(C1, nrows - off)
            if ln <= 0:
                break
            pltpu.sync_copy(xr_hbm.at[cid, pl.ds(sid * NROW + off, ln)],
                            rows_v.at[pl.ds(0, ln)])
            pltpu.sync_copy(rows_v.at[pl.ds(0, ln)],
                            acc.at[pl.ds(sid * NROW + off, ln)])

    @pl.when(sid < NTILE - 1)
    def _():
        _init(NROW)

    @pl.when(sid == NTILE - 1)
    def _():
        _init(NROW_LAST)
    plsc.subcore_barrier()

    def _chunk(k, _):
        base = (sid + k * NTILE) * C1
        stg = [pltpu.async_copy(src_hbm.at[pl.ds(base, C1)], src_v, sem),
               pltpu.async_copy(dst_hbm.at[pl.ds(base, C1)], dst_v, sem),
               pltpu.async_copy(typ_hbm.at[pl.ds(base, C1)], typ_v, sem),
               pltpu.async_copy(scl_hbm.at[pl.ds(base, C1)], scl_v, sem)]
        for d_ in stg:
            d_.wait()

        def _idx(j, _):
            def _grp(u, _):
                off = j * 128 + u * 16
                s = src_v[pl.ds(off, 16)]
                d = dst_v[pl.ds(off, 16)]
                t = typ_v[pl.ds(off, 16)]
                flat2d[j, pl.ds(u * 16, 16)] = s * R + t + tbl_off
                dst2d[j, pl.ds(u * 16, 16)] = d
                return 0
            lax.fori_loop(0, 8, _grp, 0)
            return 0
        lax.fori_loop(0, C1_ROWS, _idx, 0)

        gds = [pltpu.async_copy(xw_hbm.at[flat2d.at[j]],
                                rows_v.at[pl.ds(j * 128, 128)], sem)
               for j in range(C1_ROWS)]
        sds = []
        for j in range(C1_ROWS):
            gds[j].wait()

            def _scale(g, _):
                sv = scl_v[pl.ds(g * 16, 16)]
                for u in range(16):
                    e = g * 16 + u
                    s = sv[u]
                    rows_v[e, pl.ds(0, 16)] = rows_v[e, pl.ds(0, 16)] * s
                    rows_v[e, pl.ds(16, 16)] = rows_v[e, pl.ds(16, 16)] * s
                return 0
            lax.fori_loop(j * 8, (j + 1) * 8, _scale, 0)
            sds.append(pltpu.async_copy(rows_v.at[pl.ds(j * 128, 128)],
                                        acc.at[dst2d.at[j]], sem2, add=True))
        for d_ in sds:
            d_.wait()
        return 0

    n_mine = C1_CHUNKS // NTILE + jnp.where(sid < (C1_CHUNKS % NTILE), 1, 0)
    lax.fori_loop(0, n_mine, _chunk, 0)
    plsc.subcore_barrier()

    def _wout(nrows):
        for off in range(0, 3200, C1):
            ln = min(C1, nrows - off)
            if ln <= 0:
                break
            pltpu.sync_copy(acc.at[pl.ds(sid * NROW + off, ln)],
                            rows_v.at[pl.ds(0, ln)])
            pltpu.sync_copy(rows_v.at[pl.ds(0, ln)],
                            h_hbm.at[cid, pl.ds(sid * NROW + off, ln)])

    @pl.when(sid < NTILE - 1)
    def _():
        _wout(NROW)

    @pl.when(sid == NTILE - 1)
    def _():
        _wout(NROW_LAST)


@functools.partial(
    pl.kernel,
    out_type=jax.ShapeDtypeStruct((NSC, N, HID // 2), jnp.float32),
    mesh=_SC_MESH,
    compiler_params=_SC_PARAMS,
    scratch_types=[
        pltpu.VMEM((C1,), jnp.int32),
        pltpu.VMEM((C1,), jnp.int32),
        pltpu.VMEM((C1,), jnp.int32),
        pltpu.VMEM((C1,), jnp.float32),
        pltpu.VMEM((C1_ROWS, 128), jnp.int32),
        pltpu.VMEM((C1_ROWS, 128), jnp.int32),
        pltpu.VMEM((C1, HID // 2), jnp.float32),
        pltpu.VMEM_SHARED((N, HID // 2), jnp.float32),
        pltpu.SemaphoreType.DMA,
        pltpu.SemaphoreType.DMA,
    ],
)
def _sc_msg1(src_hbm, dst_hbm, typ_hbm, scl_hbm, xw_hbm, xr_hbm, h_hbm,
             src_v, dst_v, typ_v, scl_v, flat2d, dst2d, rows_v, acc, sem,
             sem2):
    _msg1_body(src_hbm, dst_hbm, typ_hbm, scl_hbm, xw_hbm, xr_hbm, h_hbm,
               src_v, dst_v, typ_v, scl_v, flat2d, dst2d, rows_v, acc, sem,
               sem2)


C2 = 3200
C2_ROWS = C2 // 128
C2_CHUNKS = E // NSC // C2


def _msg2_body(src_hbm, dst_hbm, typ_hbm, scl_hbm, hw_hbm, hr_hbm, p_hbm,
               src_v, dst_v, typ_v, scl_v, flat2d, dst2d, rows_v, acc, sem,
               sem2):
    cid = lax.axis_index("c")
    sid = lax.axis_index("s")

    @pl.when(cid == 0)
    def _():
        @pl.when(sid < NTILE - 1)
        def _():
            pltpu.sync_copy(hr_hbm.at[pl.ds(sid * NROW, NROW)],
                            rows_v.at[pl.ds(0, NROW)])
            pltpu.sync_copy(rows_v.at[pl.ds(0, NROW)],
                            acc.at[pl.ds(sid * NROW, NROW)])

        @pl.when(sid == NTILE - 1)
        def _():
            pltpu.sync_copy(hr_hbm.at[pl.ds(15 * NROW, NROW_LAST)],
                            rows_v.at[pl.ds(0, NROW_LAST)])
            pltpu.sync_copy(rows_v.at[pl.ds(0, NROW_LAST)],
                            acc.at[pl.ds(15 * NROW, NROW_LAST)])

    @pl.when(cid == 1)
    def _():
        def _z(i, _):
            rows_v[i, pl.ds(0, 16)] = jnp.zeros((16,), jnp.float32)
            return 0
        lax.fori_loop(0, NROW, _z, 0)

        @pl.when(sid < NTILE - 1)
        def _():
            pltpu.sync_copy(rows_v.at[pl.ds(0, NROW)],
                            acc.at[pl.ds(sid * NROW, NROW)])

        @pl.when(sid == NTILE - 1)
        def _():
            pltpu.sync_copy(rows_v.at[pl.ds(0, NROW_LAST)],
                            acc.at[pl.ds(15 * NROW, NROW_LAST)])
    plsc.subcore_barrier()

    def _chunk(k, _):
        base = (cid * (E // NSC)) + (sid + k * NTILE) * C2
        stg = [pltpu.async_copy(src_hbm.at[pl.ds(base, C2)], src_v, sem),
               pltpu.async_copy(dst_hbm.at[pl.ds(base, C2)], dst_v, sem),
               pltpu.async_copy(typ_hbm.at[pl.ds(base, C2)], typ_v, sem),
               pltpu.async_copy(scl_hbm.at[pl.ds(base, C2)], scl_v, sem)]
        for d_ in stg:
            d_.wait()

        def _idx(j, _):
            def _grp(u, _):
                off = j * 128 + u * 16
                s = src_v[pl.ds(off, 16)]
                d = dst_v[pl.ds(off, 16)]
                t = typ_v[pl.ds(off, 16)]
                flat2d[j, pl.ds(u * 16, 16)] = s * R + t
                dst2d[j, pl.ds(u * 16, 16)] = d
                return 0
            lax.fori_loop(0, 8, _grp, 0)
            return 0
        lax.fori_loop(0, C2_ROWS, _idx, 0)

        gds = [pltpu.async_copy(hw_hbm.at[flat2d.at[j]],
                                rows_v.at[pl.ds(j * 128, 128)], sem)
               for j in range(C2_ROWS)]
        sds = []
        for j in range(C2_ROWS):
            gds[j].wait()

            def _scale(g, _):
                sv = scl_v[pl.ds(g * 16, 16)]
                for u in range(16):
                    e = g * 16 + u
                    rows_v[e, pl.ds(0, 16)] = rows_v[e, pl.ds(0, 16)] * sv[u]
                return 0
            lax.fori_loop(j * 8, (j + 1) * 8, _scale, 0)
            sds.append(pltpu.async_copy(rows_v.at[pl.ds(j * 128, 128)],
                                        acc.at[dst2d.at[j]], sem2, add=True))
        for d_ in sds:
            d_.wait()
        return 0

    n_mine = C2_CHUNKS // NTILE + jnp.where(sid < (C2_CHUNKS % NTILE), 1, 0)
    lax.fori_loop(0, n_mine, _chunk, 0)
    plsc.subcore_barrier()

    @pl.when(sid < NTILE - 1)
    def _():
        pltpu.sync_copy(acc.at[pl.ds(sid * NROW, NROW)],
                        rows_v.at[pl.ds(0, NROW)])
        pltpu.sync_copy(rows_v.at[pl.ds(0, NROW)],
                        p_hbm.at[cid, pl.ds(sid * NROW, NROW)])

    @pl.when(sid == NTILE - 1)
    def _():
        pltpu.sync_copy(acc.at[pl.ds(15 * NROW, NROW_LAST)],
                        rows_v.at[pl.ds(0, NROW_LAST)])
        pltpu.sync_copy(rows_v.at[pl.ds(0, NROW_LAST)],
                        p_hbm.at[cid, pl.ds(15 * NROW, NROW_LAST)])


@functools.partial(
    pl.kernel,
    out_type=jax.ShapeDtypeStruct((NSC, N, LABELS), jnp.float32),
    mesh=_SC_MESH,
    compiler_params=_SC_PARAMS,
    scratch_types=[
        pltpu.VMEM((C2,), jnp.int32),
        pltpu.VMEM((C2,), jnp.int32),
        pltpu.VMEM((C2,), jnp.int32),
        pltpu.VMEM((C2,), jnp.float32),
        pltpu.VMEM((C2_ROWS, 128), jnp.int32),
        pltpu.VMEM((C2_ROWS, 128), jnp.int32),
        pltpu.VMEM((C2, LABELS), jnp.float32),
        pltpu.VMEM_SHARED((N, LABELS), jnp.float32),
        pltpu.SemaphoreType.DMA,
        pltpu.SemaphoreType.DMA,
    ],
)
def _sc_msg2(src_hbm, dst_hbm, typ_hbm, scl_hbm, hw_hbm, hr_hbm, p_hbm,
             src_v, dst_v, typ_v, scl_v, flat2d, dst2d, rows_v, acc, sem,
             sem2):
    _msg2_body(src_hbm, dst_hbm, typ_hbm, scl_hbm, hw_hbm, hr_hbm, p_hbm,
               src_v, dst_v, typ_v, scl_v, flat2d, dst2d, rows_v, acc, sem,
               sem2)


def _dense1_body(emb_ref, w1t_ref, b1_ref, w2t_ref, b2_ref, wr1a_ref,
                 wr1b_ref, roota_ref, rootb_ref, ba_ref, bb_ref,
                 xw_ref, xr_ref):
    e = emb_ref[...]
    x = jax.nn.sigmoid(
        jnp.dot(e, w1t_ref[...], preferred_element_type=jnp.float32)
        + b1_ref[...])
    x = jax.nn.sigmoid(
        jnp.dot(x, w2t_ref[...], preferred_element_type=jnp.float32)
        + b2_ref[...])
    xw_ref[0] = jnp.dot(x, wr1a_ref[...], preferred_element_type=jnp.float32)
    xw_ref[1] = jnp.dot(x, wr1b_ref[...], preferred_element_type=jnp.float32)
    xr_ref[0] = (jnp.dot(x, roota_ref[...], preferred_element_type=jnp.float32)
                 + ba_ref[...])
    xr_ref[1] = (jnp.dot(x, rootb_ref[...], preferred_element_type=jnp.float32)
                 + bb_ref[...])


def _dense1(emb, W1t, b1, W2t, b2, Wr1a, Wr1b, roota, rootb, ba, bb):
    grid = (N // ROW_BLK,)
    return pl.pallas_call(
        _dense1_body,
        grid=grid,
        in_specs=[
            pl.BlockSpec((ROW_BLK, EMB), lambda i: (i, 0)),
            pl.BlockSpec((EMB, OUT_F), lambda i: (0, 0)),
            pl.BlockSpec((1, OUT_F), lambda i: (0, 0)),
            pl.BlockSpec((OUT_F, EMB), lambda i: (0, 0)),
            pl.BlockSpec((1, EMB), lambda i: (0, 0)),
            pl.BlockSpec((EMB, R * HID // 2), lambda i: (0, 0)),
            pl.BlockSpec((EMB, R * HID // 2), lambda i: (0, 0)),
            pl.BlockSpec((EMB, HID // 2), lambda i: (0, 0)),
            pl.BlockSpec((EMB, HID // 2), lambda i: (0, 0)),
            pl.BlockSpec((1, HID // 2), lambda i: (0, 0)),
            pl.BlockSpec((1, HID // 2), lambda i: (0, 0)),
        ],
        out_specs=[
            pl.BlockSpec((NSC, ROW_BLK, R * HID // 2), lambda i: (0, i, 0)),
            pl.BlockSpec((NSC, ROW_BLK, HID // 2), lambda i: (0, i, 0)),
        ],
        out_shape=[
            jax.ShapeDtypeStruct((NSC, N, R * HID // 2), jnp.float32),
            jax.ShapeDtypeStruct((NSC, N, HID // 2), jnp.float32),
        ],
    )(emb, W1t, b1, W2t, b2, Wr1a, Wr1b, roota, rootb, ba, bb)


def _dense2_body(ha_ref, hb_ref, wr2a_ref, wr2b_ref, root2a_ref, root2b_ref,
                 bias2_ref, hw_ref, hr_ref):
    ha = jax.nn.relu(ha_ref[...])
    hb = jax.nn.relu(hb_ref[...])
    hw_ref[...] = (
        jnp.dot(ha, wr2a_ref[...], preferred_element_type=jnp.float32)
        + jnp.dot(hb, wr2b_ref[...], preferred_element_type=jnp.float32))
    hr_ref[...] = (
        jnp.dot(ha, root2a_ref[...], preferred_element_type=jnp.float32)
        + jnp.dot(hb, root2b_ref[...], preferred_element_type=jnp.float32)
        + bias2_ref[...])


def _dense2(ha, hb, Wr2a, Wr2b, root2a, root2b, bias2):
    grid = (N // ROW_BLK,)
    return pl.pallas_call(
        _dense2_body,
        grid=grid,
        in_specs=[
            pl.BlockSpec((ROW_BLK, HID // 2), lambda i: (i, 0)),
            pl.BlockSpec((ROW_BLK, HID // 2), lambda i: (i, 0)),
            pl.BlockSpec((HID // 2, R * LABELS), lambda i: (0, 0)),
            pl.BlockSpec((HID // 2, R * LABELS), lambda i: (0, 0)),
            pl.BlockSpec((HID // 2, LABELS), lambda i: (0, 0)),
            pl.BlockSpec((HID // 2, LABELS), lambda i: (0, 0)),
            pl.BlockSpec((1, LABELS), lambda i: (0, 0)),
        ],
        out_specs=[
            pl.BlockSpec((ROW_BLK, R * LABELS), lambda i: (i, 0)),
            pl.BlockSpec((ROW_BLK, LABELS), lambda i: (i, 0)),
        ],
        out_shape=[
            jax.ShapeDtypeStruct((N, R * LABELS), jnp.float32),
            jax.ShapeDtypeStruct((N, LABELS), jnp.float32),
        ],
    )(ha, hb, Wr2a, Wr2b, root2a, root2b, bias2)


def _final_body(p_ref, out_ref):
    out_ref[...] = jax.nn.sigmoid(p_ref[0] + p_ref[1])


def _final(p):
    grid = (N // ROW_BLK,)
    return pl.pallas_call(
        _final_body,
        grid=grid,
        in_specs=[pl.BlockSpec((NSC, ROW_BLK, LABELS), lambda i: (0, i, 0))],
        out_specs=pl.BlockSpec((ROW_BLK, LABELS), lambda i: (i, 0)),
        out_shape=jax.ShapeDtypeStruct((N, LABELS), jnp.float32),
    )(p)


def kernel(emb, W1, b1, W2, b2, Wr1, root1, bias1, Wr2, root2, bias2,
           edge_index, edge_type):
    src = edge_index[0]
    dst = edge_index[1]

    edge_scale = _sc_counts(dst, edge_type)

    W1t = W1.T
    W2t = W2.T
    Wr1f = Wr1.transpose(1, 0, 2)
    Wr1a = Wr1f[:, :, :HID // 2].reshape(EMB, R * HID // 2)
    Wr1b = Wr1f[:, :, HID // 2:].reshape(EMB, R * HID // 2)
    roota = root1[:, :HID // 2]
    rootb = root1[:, HID // 2:]

    xw, xr = _dense1(emb, W1t, b1.reshape(1, -1), W2t, b2.reshape(1, -1),
                     Wr1a, Wr1b, roota, rootb,
                     bias1[:HID // 2].reshape(1, -1),
                     bias1[HID // 2:].reshape(1, -1))

    h_raw = _sc_msg1(src, dst, edge_type, edge_scale,
                     xw.reshape(NSC * NR, HID // 2), xr)

    Wr2f = Wr2.transpose(1, 0, 2)
    Wr2a = Wr2f[:HID // 2].reshape(HID // 2, R * LABELS)
    Wr2b = Wr2f[HID // 2:].reshape(HID // 2, R * LABELS)
    root2a = root2[:HID // 2]
    root2b = root2[HID // 2:]

    hw, hr = _dense2(h_raw[0], h_raw[1], Wr2a, Wr2b, root2a, root2b,
                     bias2.reshape(1, -1))

    p = _sc_msg2(src, dst, edge_type, edge_scale, hw.reshape(NR, LABELS), hr)

    return _final(p)


# speedup vs baseline: 19.1330x; 1.0257x over previous
import functools
import jax
import jax.numpy as jnp
from jax import lax
from jax.experimental import pallas as pl
from jax.experimental.pallas import tpu as pltpu
from jax.experimental.pallas import tpu_sc as plsc

N = 50000
E = 800000
R = 8
EMB = 64
HID = 64
LABELS = 16
OUT_F = 112

ROW_BLK = 400

NSC = 2
NTILE = 16
NR = N * R
SLICE_W = 25024
NR_PAD = NTILE * SLICE_W
NROW = 3128
NROW_LAST = N - 15 * NROW

_SC_MESH = plsc.VectorSubcoreMesh(
    core_axis_name="c", subcore_axis_name="s",
    num_cores=NSC, num_subcores=NTILE)
_SC_PARAMS = pltpu.CompilerParams(use_tc_tiling_on_sc=False)


C_CNT = 3200
CNT_ROWS = C_CNT // 128
CNT_CHUNKS = E // C_CNT
SCL_CHUNKS = E // NSC // C_CNT


def _counts_body(dst_hbm, typ_hbm, scl_hbm, dst_v, typ_v, comp_v, ones_v,
                 sweep_v, invv, table, sem, sem2):
    cid = lax.axis_index("c")
    sid = lax.axis_index("s")

    def _fill(i, _):
        ones_v[pl.ds(i * 16, 16)] = jnp.full((16,), 1.0, jnp.float32)
        return 0
    lax.fori_loop(0, 128 // 16, _fill, 0)

    def _zero(i, _):
        sweep_v[pl.ds(i * 16, 16)] = jnp.zeros((16,), jnp.float32)
        return 0
    lax.fori_loop(0, SLICE_W // 16, _zero, 0)

    pltpu.sync_copy(sweep_v, table.at[pl.ds(sid * SLICE_W, SLICE_W)])
    plsc.subcore_barrier()

    def _chunk(k, _):
        base = (sid + k * NTILE) * C_CNT
        stg = [pltpu.async_copy(dst_hbm.at[pl.ds(base, C_CNT)], dst_v, sem),
               pltpu.async_copy(typ_hbm.at[pl.ds(base, C_CNT)], typ_v, sem)]
        for d_ in stg:
            d_.wait()

        def _comp(j, _):
            def _grp(u, _):
                off = j * 128 + u * 16
                d = dst_v[pl.ds(off, 16)]
                t = typ_v[pl.ds(off, 16)]
                comp_v[j, pl.ds(u * 16, 16)] = d * R + t
                return 0
            lax.fori_loop(0, 8, _grp, 0)
            return 0
        lax.fori_loop(0, CNT_ROWS, _comp, 0)

        scds = [pltpu.async_copy(ones_v, table.at[comp_v.at[j]], sem2,
                                 add=True)
                for j in range(CNT_ROWS)]
        for d_ in scds:
            d_.wait()
        return 0

    n_mine = CNT_CHUNKS // NTILE + jnp.where(sid < (CNT_CHUNKS % NTILE), 1, 0)
    lax.fori_loop(0, n_mine, _chunk, 0)
    plsc.subcore_barrier()

    pltpu.sync_copy(table.at[pl.ds(sid * SLICE_W, SLICE_W)], sweep_v)

    def _inv(i, _):
        c = sweep_v[pl.ds(i * 16, 16)]
        sweep_v[pl.ds(i * 16, 16)] = jnp.where(
            c > 0.0, 1.0 / jnp.maximum(c, 1.0), 0.0)
        return 0
    lax.fori_loop(0, SLICE_W // 16, _inv, 0)
    pltpu.sync_copy(sweep_v, table.at[pl.ds(sid * SLICE_W, SLICE_W)])
    plsc.subcore_barrier()

    def _schunk(k, _):
        base = cid * (E // NSC) + (sid + k * NTILE) * C_CNT
        pltpu.sync_copy(dst_hbm.at[pl.ds(base, C_CNT)], dst_v)
        pltpu.sync_copy(typ_hbm.at[pl.ds(base, C_CNT)], typ_v)

        def _comp(j, _):
            def _grp(u, _):
                off = j * 128 + u * 16
                d = dst_v[pl.ds(off, 16)]
                t = typ_v[pl.ds(off, 16)]
                comp_v[j, pl.ds(u * 16, 16)] = d * R + t
                return 0
            lax.fori_loop(0, 8, _grp, 0)
            return 0
        lax.fori_loop(0, CNT_ROWS, _comp, 0)

        gds = [pltpu.async_copy(table.at[comp_v.at[j]],
                                invv.at[pl.ds(j * 128, 128)], sem2)
               for j in range(CNT_ROWS)]
        for d_ in gds:
            d_.wait()
        pltpu.sync_copy(invv, scl_hbm.at[pl.ds(base, C_CNT)])
        return 0

    n_mine2 = SCL_CHUNKS // NTILE + jnp.where(sid < (SCL_CHUNKS % NTILE), 1, 0)
    lax.fori_loop(0, n_mine2, _schunk, 0)


@functools.partial(
    pl.kernel,
    out_type=jax.ShapeDtypeStruct((E,), jnp.float32),
    mesh=_SC_MESH,
    compiler_params=_SC_PARAMS,
    scratch_types=[
        pltpu.VMEM((C_CNT,), jnp.int32),
        pltpu.VMEM((C_CNT,), jnp.int32),
        pltpu.VMEM((CNT_ROWS, 128), jnp.int32),
        pltpu.VMEM((128,), jnp.float32),
        pltpu.VMEM((SLICE_W,), jnp.float32),
        pltpu.VMEM((C_CNT,), jnp.float32),
        pltpu.VMEM_SHARED((NR_PAD,), jnp.float32),
        pltpu.SemaphoreType.DMA,
        pltpu.SemaphoreType.DMA,
    ],
)
def _sc_counts(dst_hbm, typ_hbm, scl_hbm, dst_v, typ_v, comp_v, ones_v,
               sweep_v, invv, table, sem, sem2):
    _counts_body(dst_hbm, typ_hbm, scl_hbm, dst_v, typ_v, comp_v, ones_v,
                 sweep_v, invv, table, sem, sem2)


C1 = 640
C1_ROWS = C1 // 128
C1_CHUNKS = E // C1


def _msg1_body(src_hbm, dst_hbm, typ_hbm, scl_hbm, xw_hbm, xr_hbm, h_hbm,
               src_v, dst_v, typ_v, scl_v, flat2d, dst2d, rows_v, acc, sem,
               sem2):
    cid = lax.axis_index("c")
    sid = lax.axis_index("s")
    tbl_off = cid * NR

    def _init(nrows):
        for off in range(0, 3200, C1):
            ln = min(C1, nrows - off)
            if ln <= 0:
                break
            pltpu.sync_copy(xr_hbm.at[cid, pl.ds(sid * NROW + off, ln)],
                            rows_v.at[pl.ds(0, ln)])
            pltpu.sync_copy(rows_v.at[pl.ds(0, ln)],
                            acc.at[pl.ds(sid * NROW + off, ln)])

    @pl.when(sid < NTILE - 1)
    def _():
        _init(NROW)

    @pl.when(sid == NTILE - 1)
    def _():
        _init(NROW_LAST)
    plsc.subcore_barrier()

    def _chunk(k, _):
        base = (sid + k * NTILE) * C1
        stg = [pltpu.async_copy(src_hbm.at[pl.ds(base, C1)], src_v, sem),
               pltpu.async_copy(dst_hbm.at[pl.ds(base, C1)], dst_v, sem),
               pltpu.async_copy(typ_hbm.at[pl.ds(base, C1)], typ_v, sem),
               pltpu.async_copy(scl_hbm.at[pl.ds(base, C1)], scl_v, sem)]
        for d_ in stg:
            d_.wait()

        def _idx(j, _):
            def _grp(u, _):
                off = j * 128 + u * 16
                s = src_v[pl.ds(off, 16)]
                d = dst_v[pl.ds(off, 16)]
                t = typ_v[pl.ds(off, 16)]
                flat2d[j, pl.ds(u * 16, 16)] = s * R + t + tbl_off
                dst2d[j, pl.ds(u * 16, 16)] = d
                return 0
            lax.fori_loop(0, 8, _grp, 0)
            return 0
        lax.fori_loop(0, C1_ROWS, _idx, 0)

        gds = [pltpu.async_copy(xw_hbm.at[flat2d.at[j]],
                                rows_v.at[pl.ds(j * 128, 128)], sem)
               for j in range(C1_ROWS)]
        sds = []
        for j in range(C1_ROWS):
            gds[j].wait()

            def _scale(g, _):
                sv = scl_v[pl.ds(g * 16, 16)]
                for u in range(16):
                    e = g * 16 + u
                    s = sv[u]
                    rows_v[e, pl.ds(0, 16)] = rows_v[e, pl.ds(0, 16)] * s
                    rows_v[e, pl.ds(16, 16)] = rows_v[e, pl.ds(16, 16)] * s
                return 0
            lax.fori_loop(j * 8, (j + 1) * 8, _scale, 0)
            sds.append(pltpu.async_copy(rows_v.at[pl.ds(j * 128, 128)],
                                        acc.at[dst2d.at[j]], sem2, add=True))
        for d_ in sds:
            d_.wait()
        return 0

    n_mine = C1_CHUNKS // NTILE + jnp.where(sid < (C1_CHUNKS % NTILE), 1, 0)
    lax.fori_loop(0, n_mine, _chunk, 0)
    plsc.subcore_barrier()

    def _wout(nrows):
        for off in range(0, 3200, C1):
            ln = min(C1, nrows - off)
            if ln <= 0:
                break
            pltpu.sync_copy(acc.at[pl.ds(sid * NROW + off, ln)],
                            rows_v.at[pl.ds(0, ln)])
            pltpu.sync_copy(rows_v.at[pl.ds(0, ln)],
                            h_hbm.at[cid, pl.ds(sid * NROW + off, ln)])

    @pl.when(sid < NTILE - 1)
    def _():
        _wout(NROW)

    @pl.when(sid == NTILE - 1)
    def _():
        _wout(NROW_LAST)


@functools.partial(
    pl.kernel,
    out_type=jax.ShapeDtypeStruct((NSC, N, HID // 2), jnp.float32),
    mesh=_SC_MESH,
    compiler_params=_SC_PARAMS,
    scratch_types=[
        pltpu.VMEM((C1,), jnp.int32),
        pltpu.VMEM((C1,), jnp.int32),
        pltpu.VMEM((C1,), jnp.int32),
        pltpu.VMEM((C1,), jnp.float32),
        pltpu.VMEM((C1_ROWS, 128), jnp.int32),
        pltpu.VMEM((C1_ROWS, 128), jnp.int32),
        pltpu.VMEM((C1, HID // 2), jnp.float32),
        pltpu.VMEM_SHARED((N, HID // 2), jnp.float32),
        pltpu.SemaphoreType.DMA,
        pltpu.SemaphoreType.DMA,
    ],
)
def _sc_msg1(src_hbm, dst_hbm, typ_hbm, scl_hbm, xw_hbm, xr_hbm, h_hbm,
             src_v, dst_v, typ_v, scl_v, flat2d, dst2d, rows_v, acc, sem,
             sem2):
    _msg1_body(src_hbm, dst_hbm, typ_hbm, scl_hbm, xw_hbm, xr_hbm, h_hbm,
               src_v, dst_v, typ_v, scl_v, flat2d, dst2d, rows_v, acc, sem,
               sem2)


C2 = 3200
C2_ROWS = C2 // 128
C2_CHUNKS = E // NSC // C2


def _msg2_body(src_hbm, dst_hbm, typ_hbm, scl_hbm, hw_hbm, hr_hbm, p_hbm,
               src_v, dst_v, typ_v, scl_v, flat2d, dst2d, rows_v, acc, sem,
               sem2):
    cid = lax.axis_index("c")
    sid = lax.axis_index("s")

    @pl.when(cid == 0)
    def _():
        @pl.when(sid < NTILE - 1)
        def _():
            pltpu.sync_copy(hr_hbm.at[pl.ds(sid * NROW, NROW)],
                            rows_v.at[pl.ds(0, NROW)])
            pltpu.sync_copy(rows_v.at[pl.ds(0, NROW)],
                            acc.at[pl.ds(sid * NROW, NROW)])

        @pl.when(sid == NTILE - 1)
        def _():
            pltpu.sync_copy(hr_hbm.at[pl.ds(15 * NROW, NROW_LAST)],
                            rows_v.at[pl.ds(0, NROW_LAST)])
            pltpu.sync_copy(rows_v.at[pl.ds(0, NROW_LAST)],
                            acc.at[pl.ds(15 * NROW, NROW_LAST)])

    @pl.when(cid == 1)
    def _():
        def _z(i, _):
            rows_v[i, pl.ds(0, 16)] = jnp.zeros((16,), jnp.float32)
            return 0
        lax.fori_loop(0, NROW, _z, 0)

        @pl.when(sid < NTILE - 1)
        def _():
            pltpu.sync_copy(rows_v.at[pl.ds(0, NROW)],
                            acc.at[pl.ds(sid * NROW, NROW)])

        @pl.when(sid == NTILE - 1)
        def _():
            pltpu.sync_copy(rows_v.at[pl.ds(0, NROW_LAST)],
                            acc.at[pl.ds(15 * NROW, NROW_LAST)])
    plsc.subcore_barrier()

    def _chunk(k, _):
        base = (cid * (E // NSC)) + (sid + k * NTILE) * C2
        stg = [pltpu.async_copy(src_hbm.at[pl.ds(base, C2)], src_v, sem),
               pltpu.async_copy(dst_hbm.at[pl.ds(base, C2)], dst_v, sem),
               pltpu.async_copy(typ_hbm.at[pl.ds(base, C2)], typ_v, sem),
               pltpu.async_copy(scl_hbm.at[pl.ds(base, C2)], scl_v, sem)]
        for d_ in stg:
            d_.wait()

        def _idx(j, _):
            def _grp(u, _):
                off = j * 128 + u * 16
                s = src_v[pl.ds(off, 16)]
                d = dst_v[pl.ds(off, 16)]
                t = typ_v[pl.ds(off, 16)]
                flat2d[j, pl.ds(u * 16, 16)] = s * R + t
                dst2d[j, pl.ds(u * 16, 16)] = d
                return 0
            lax.fori_loop(0, 8, _grp, 0)
            return 0
        lax.fori_loop(0, C2_ROWS, _idx, 0)

        gds = [pltpu.async_copy(hw_hbm.at[flat2d.at[j]],
                                rows_v.at[pl.ds(j * 128, 128)], sem)
               for j in range(C2_ROWS)]
        sds = []
        for j in range(C2_ROWS):
            gds[j].wait()

            def _scale(g, _):
                sv = scl_v[pl.ds(g * 16, 16)]
                for u in range(16):
                    e = g * 16 + u
                    rows_v[e, pl.ds(0, 16)] = rows_v[e, pl.ds(0, 16)] * sv[u]
                return 0
            lax.fori_loop(j * 8, (j + 1) * 8, _scale, 0)
            sds.append(pltpu.async_copy(rows_v.at[pl.ds(j * 128, 128)],
                                        acc.at[dst2d.at[j]], sem2, add=True))
        for d_ in sds:
            d_.wait()
        return 0

    n_mine = C2_CHUNKS // NTILE + jnp.where(sid < (C2_CHUNKS % NTILE), 1, 0)
    lax.fori_loop(0, n_mine, _chunk, 0)
    plsc.subcore_barrier()

    @pl.when(sid < NTILE - 1)
    def _():
        pltpu.sync_copy(acc.at[pl.ds(sid * NROW, NROW)],
                        rows_v.at[pl.ds(0, NROW)])
        pltpu.sync_copy(rows_v.at[pl.ds(0, NROW)],
                        p_hbm.at[cid, pl.ds(sid * NROW, NROW)])

    @pl.when(sid == NTILE - 1)
    def _():
        pltpu.sync_copy(acc.at[pl.ds(15 * NROW, NROW_LAST)],
                        rows_v.at[pl.ds(0, NROW_LAST)])
        pltpu.sync_copy(rows_v.at[pl.ds(0, NROW_LAST)],
                        p_hbm.at[cid, pl.ds(15 * NROW, NROW_LAST)])


@functools.partial(
    pl.kernel,
    out_type=jax.ShapeDtypeStruct((NSC, N, LABELS), jnp.float32),
    mesh=_SC_MESH,
    compiler_params=_SC_PARAMS,
    scratch_types=[
        pltpu.VMEM((C2,), jnp.int32),
        pltpu.VMEM((C2,), jnp.int32),
        pltpu.VMEM((C2,), jnp.int32),
        pltpu.VMEM((C2,), jnp.float32),
        pltpu.VMEM((C2_ROWS, 128), jnp.int32),
        pltpu.VMEM((C2_ROWS, 128), jnp.int32),
        pltpu.VMEM((C2, LABELS), jnp.float32),
        pltpu.VMEM_SHARED((N, LABELS), jnp.float32),
        pltpu.SemaphoreType.DMA,
        pltpu.SemaphoreType.DMA,
    ],
)
def _sc_msg2(src_hbm, dst_hbm, typ_hbm, scl_hbm, hw_hbm, hr_hbm, p_hbm,
             src_v, dst_v, typ_v, scl_v, flat2d, dst2d, rows_v, acc, sem,
             sem2):
    _msg2_body(src_hbm, dst_hbm, typ_hbm, scl_hbm, hw_hbm, hr_hbm, p_hbm,
               src_v, dst_v, typ_v, scl_v, flat2d, dst2d, rows_v, acc, sem,
               sem2)


def _dense1_body(emb_ref, w1t_ref, b1_ref, w2t_ref, b2_ref, wr1_ref,
                 root_ref, bias_ref, xw_ref, xr_ref):
    e = emb_ref[...]
    x = jax.nn.sigmoid(
        jnp.dot(e, w1t_ref[...], preferred_element_type=jnp.float32)
        + b1_ref[...])
    x = jax.nn.sigmoid(
        jnp.dot(x, w2t_ref[...], preferred_element_type=jnp.float32)
        + b2_ref[...])
    xw_ref[...] = jnp.dot(
        x, wr1_ref[0], preferred_element_type=jnp.float32).reshape(
            ROW_BLK * 2, 128)
    xr_ref[0] = (jnp.dot(x, root_ref[0], preferred_element_type=jnp.float32)
                 + bias_ref[0])


def _dense1(emb, W1t, b1, W2t, b2, Wr1ab, rootab, biasab):
    grid = (N // ROW_BLK, NSC)
    return pl.pallas_call(
        _dense1_body,
        grid=grid,
        in_specs=[
            pl.BlockSpec((ROW_BLK, EMB), lambda i, h: (i, 0)),
            pl.BlockSpec((EMB, OUT_F), lambda i, h: (0, 0)),
            pl.BlockSpec((1, OUT_F), lambda i, h: (0, 0)),
            pl.BlockSpec((OUT_F, EMB), lambda i, h: (0, 0)),
            pl.BlockSpec((1, EMB), lambda i, h: (0, 0)),
            pl.BlockSpec((1, EMB, R * HID // 2), lambda i, h: (h, 0, 0)),
            pl.BlockSpec((1, EMB, HID // 2), lambda i, h: (h, 0, 0)),
            pl.BlockSpec((1, 1, HID // 2), lambda i, h: (h, 0, 0)),
        ],
        out_specs=[
            pl.BlockSpec((ROW_BLK * 2, 128),
                         lambda i, h: (h * (N // ROW_BLK) + i, 0)),
            pl.BlockSpec((1, ROW_BLK, HID // 2), lambda i, h: (h, i, 0)),
        ],
        out_shape=[
            jax.ShapeDtypeStruct((NSC * NR * (HID // 2) // 128, 128),
                                 jnp.float32),
            jax.ShapeDtypeStruct((NSC, N, HID // 2), jnp.float32),
        ],
    )(emb, W1t, b1, W2t, b2, Wr1ab, rootab, biasab)


def _dense2_body(ha_ref, hb_ref, wr2a_ref, wr2b_ref, root2a_ref, root2b_ref,
                 bias2_ref, hw_ref, hr_ref):
    ha = jax.nn.relu(ha_ref[...])
    hb = jax.nn.relu(hb_ref[...])
    hw_ref[...] = (
        jnp.dot(ha, wr2a_ref[...], preferred_element_type=jnp.float32)
        + jnp.dot(hb, wr2b_ref[...], preferred_element_type=jnp.float32))
    hr_ref[...] = (
        jnp.dot(ha, root2a_ref[...], preferred_element_type=jnp.float32)
        + jnp.dot(hb, root2b_ref[...], preferred_element_type=jnp.float32)
        + bias2_ref[...])


def _dense2(ha, hb, Wr2a, Wr2b, root2a, root2b, bias2):
    grid = (N // ROW_BLK,)
    return pl.pallas_call(
        _dense2_body,
        grid=grid,
        in_specs=[
            pl.BlockSpec((ROW_BLK, HID // 2), lambda i: (i, 0)),
            pl.BlockSpec((ROW_BLK, HID // 2), lambda i: (i, 0)),
            pl.BlockSpec((HID // 2, R * LABELS), lambda i: (0, 0)),
            pl.BlockSpec((HID // 2, R * LABELS), lambda i: (0, 0)),
            pl.BlockSpec((HID // 2, LABELS), lambda i: (0, 0)),
            pl.BlockSpec((HID // 2, LABELS), lambda i: (0, 0)),
            pl.BlockSpec((1, LABELS), lambda i: (0, 0)),
        ],
        out_specs=[
            pl.BlockSpec((ROW_BLK, R * LABELS), lambda i: (i, 0)),
            pl.BlockSpec((ROW_BLK, LABELS), lambda i: (i, 0)),
        ],
        out_shape=[
            jax.ShapeDtypeStruct((N, R * LABELS), jnp.float32),
            jax.ShapeDtypeStruct((N, LABELS), jnp.float32),
        ],
    )(ha, hb, Wr2a, Wr2b, root2a, root2b, bias2)


def _final_body(p_ref, out_ref):
    out_ref[...] = jax.nn.sigmoid(p_ref[0] + p_ref[1])


def _final(p):
    grid = (N // ROW_BLK,)
    return pl.pallas_call(
        _final_body,
        grid=grid,
        in_specs=[pl.BlockSpec((NSC, ROW_BLK, LABELS), lambda i: (0, i, 0))],
        out_specs=pl.BlockSpec((ROW_BLK, LABELS), lambda i: (i, 0)),
        out_shape=jax.ShapeDtypeStruct((N, LABELS), jnp.float32),
    )(p)


def kernel(emb, W1, b1, W2, b2, Wr1, root1, bias1, Wr2, root2, bias2,
           edge_index, edge_type):
    src = edge_index[0]
    dst = edge_index[1]

    edge_scale = _sc_counts(dst, edge_type)

    W1t = W1.T
    W2t = W2.T
    Wr1f = Wr1.transpose(1, 0, 2)
    Wr1ab = jnp.stack([Wr1f[:, :, :HID // 2].reshape(EMB, R * HID // 2),
                       Wr1f[:, :, HID // 2:].reshape(EMB, R * HID // 2)])
    rootab = jnp.stack([root1[:, :HID // 2], root1[:, HID // 2:]])
    biasab = jnp.stack([bias1[:HID // 2].reshape(1, -1),
                        bias1[HID // 2:].reshape(1, -1)])

    xw, xr = _dense1(emb, W1t, b1.reshape(1, -1), W2t, b2.reshape(1, -1),
                     Wr1ab, rootab, biasab)

    h_raw = _sc_msg1(src, dst, edge_type, edge_scale,
                     xw.reshape(NSC * NR, HID // 2), xr)

    Wr2f = Wr2.transpose(1, 0, 2)
    Wr2a = Wr2f[:HID // 2].reshape(HID // 2, R * LABELS)
    Wr2b = Wr2f[HID // 2:].reshape(HID // 2, R * LABELS)
    root2a = root2[:HID // 2]
    root2b = root2[HID // 2:]

    hw, hr = _dense2(h_raw[0], h_raw[1], Wr2a, Wr2b, root2a, root2b,
                     bias2.reshape(1, -1))

    p = _sc_msg2(src, dst, edge_type, edge_scale, hw.reshape(NR, LABELS), hr)

    return _final(p)


# speedup vs baseline: 26.3029x; 1.3747x over previous
import functools
import jax
import jax.numpy as jnp
from jax import lax
from jax.experimental import pallas as pl
from jax.experimental.pallas import tpu as pltpu
from jax.experimental.pallas import tpu_sc as plsc

N = 50000
E = 800000
R = 8
EMB = 64
HID = 64
LABELS = 16
OUT_F = 112

ROW_BLK = 400

NSC = 2
NTILE = 16
NR = N * R
SLICE_W = 25024
NR_PAD = NTILE * SLICE_W
NROW = 3128
NROW_LAST = N - 15 * NROW

_SC_MESH = plsc.VectorSubcoreMesh(
    core_axis_name="c", subcore_axis_name="s",
    num_cores=NSC, num_subcores=NTILE)
_SC_PARAMS = pltpu.CompilerParams(use_tc_tiling_on_sc=False)


C_CNT = 3200
CNT_ROWS = C_CNT // 128
CNT_CHUNKS = E // C_CNT
SCL_CHUNKS = E // NSC // C_CNT


def _counts_body(dst_hbm, typ_hbm, scl_hbm, dst_v, typ_v, comp_v, ones_v,
                 sweep_v, invv, table, sem, sem2):
    cid = lax.axis_index("c")
    sid = lax.axis_index("s")

    def _fill(i, _):
        ones_v[pl.ds(i * 16, 16)] = jnp.full((16,), 1.0, jnp.float32)
        return 0
    lax.fori_loop(0, 128 // 16, _fill, 0)

    def _zero(i, _):
        sweep_v[pl.ds(i * 16, 16)] = jnp.zeros((16,), jnp.float32)
        return 0
    lax.fori_loop(0, SLICE_W // 16, _zero, 0)

    pltpu.sync_copy(sweep_v, table.at[pl.ds(sid * SLICE_W, SLICE_W)])
    plsc.subcore_barrier()

    def _chunk(k, _):
        base = (sid + k * NTILE) * C_CNT
        stg = [pltpu.async_copy(dst_hbm.at[pl.ds(base, C_CNT)], dst_v, sem),
               pltpu.async_copy(typ_hbm.at[pl.ds(base, C_CNT)], typ_v, sem)]
        for d_ in stg:
            d_.wait()

        def _comp(j, _):
            def _grp(u, _):
                off = j * 128 + u * 16
                d = dst_v[pl.ds(off, 16)]
                t = typ_v[pl.ds(off, 16)]
                comp_v[j, pl.ds(u * 16, 16)] = d * R + t
                return 0
            lax.fori_loop(0, 8, _grp, 0)
            return 0
        lax.fori_loop(0, CNT_ROWS, _comp, 0)

        scds = [pltpu.async_copy(ones_v, table.at[comp_v.at[j]], sem2,
                                 add=True)
                for j in range(CNT_ROWS)]
        for d_ in scds:
            d_.wait()
        return 0

    n_mine = CNT_CHUNKS // NTILE + jnp.where(sid < (CNT_CHUNKS % NTILE), 1, 0)
    lax.fori_loop(0, n_mine, _chunk, 0)
    plsc.subcore_barrier()

    pltpu.sync_copy(table.at[pl.ds(sid * SLICE_W, SLICE_W)], sweep_v)

    def _inv(i, _):
        c = sweep_v[pl.ds(i * 16, 16)]
        sweep_v[pl.ds(i * 16, 16)] = jnp.where(
            c > 0.0, 1.0 / jnp.maximum(c, 1.0), 0.0)
        return 0
    lax.fori_loop(0, SLICE_W // 16, _inv, 0)
    pltpu.sync_copy(sweep_v, table.at[pl.ds(sid * SLICE_W, SLICE_W)])
    plsc.subcore_barrier()

    def _schunk(k, _):
        base = cid * (E // NSC) + (sid + k * NTILE) * C_CNT
        pltpu.sync_copy(dst_hbm.at[pl.ds(base, C_CNT)], dst_v)
        pltpu.sync_copy(typ_hbm.at[pl.ds(base, C_CNT)], typ_v)

        def _comp(j, _):
            def _grp(u, _):
                off = j * 128 + u * 16
                d = dst_v[pl.ds(off, 16)]
                t = typ_v[pl.ds(off, 16)]
                comp_v[j, pl.ds(u * 16, 16)] = d * R + t
                return 0
            lax.fori_loop(0, 8, _grp, 0)
            return 0
        lax.fori_loop(0, CNT_ROWS, _comp, 0)

        gds = [pltpu.async_copy(table.at[comp_v.at[j]],
                                invv.at[pl.ds(j * 128, 128)], sem2)
               for j in range(CNT_ROWS)]
        for d_ in gds:
            d_.wait()
        pltpu.sync_copy(invv, scl_hbm.at[pl.ds(base, C_CNT)])
        return 0

    n_mine2 = SCL_CHUNKS // NTILE + jnp.where(sid < (SCL_CHUNKS % NTILE), 1, 0)
    lax.fori_loop(0, n_mine2, _schunk, 0)


@functools.partial(
    pl.kernel,
    out_type=jax.ShapeDtypeStruct((E,), jnp.float32),
    mesh=_SC_MESH,
    compiler_params=_SC_PARAMS,
    scratch_types=[
        pltpu.VMEM((C_CNT,), jnp.int32),
        pltpu.VMEM((C_CNT,), jnp.int32),
        pltpu.VMEM((CNT_ROWS, 128), jnp.int32),
        pltpu.VMEM((128,), jnp.float32),
        pltpu.VMEM((SLICE_W,), jnp.float32),
        pltpu.VMEM((C_CNT,), jnp.float32),
        pltpu.VMEM_SHARED((NR_PAD,), jnp.float32),
        pltpu.SemaphoreType.DMA,
        pltpu.SemaphoreType.DMA,
    ],
)
def _sc_counts(dst_hbm, typ_hbm, scl_hbm, dst_v, typ_v, comp_v, ones_v,
               sweep_v, invv, table, sem, sem2):
    _counts_body(dst_hbm, typ_hbm, scl_hbm, dst_v, typ_v, comp_v, ones_v,
                 sweep_v, invv, table, sem, sem2)


C1 = 640
C1_ROWS = C1 // 128
C1_CHUNKS = E // C1


def _msg1_body(src_hbm, dst_hbm, typ_hbm, scl_hbm, xw_hbm, xr_hbm, h_hbm,
               src_v, dst_v, typ_v, scl_v, flat2d, dst2d, rows_v, acc, sem,
               sem2):
    cid = lax.axis_index("c")
    sid = lax.axis_index("s")
    tbl_off = cid * NR

    def _init(nrows):
        for off in range(0, 3200, C1):
            ln = min(C1, nrows - off)
            if ln <= 0:
                break
            pltpu.sync_copy(xr_hbm.at[cid, pl.ds(sid * NROW + off, ln)],
                            rows_v.at[pl.ds(0, ln)])
            pltpu.sync_copy(rows_v.at[pl.ds(0, ln)],
                            acc.at[pl.ds(sid * NROW + off, ln)])

    @pl.when(sid < NTILE - 1)
    def _():
        _init(NROW)

    @pl.when(sid == NTILE - 1)
    def _():
        _init(NROW_LAST)
    plsc.subcore_barrier()

    def _chunk(k, _):
        base = (sid + k * NTILE) * C1
        stg = [pltpu.async_copy(src_hbm.at[pl.ds(base, C1)], src_v, sem),
               pltpu.async_copy(dst_hbm.at[pl.ds(base, C1)], dst_v, sem),
               pltpu.async_copy(typ_hbm.at[pl.ds(base, C1)], typ_v, sem),
               pltpu.async_copy(scl_hbm.at[pl.ds(base, C1)], scl_v, sem)]
        for d_ in stg:
            d_.wait()

        def _idx(j, _):
            def _grp(u, _):
                off = j * 128 + u * 16
                s = src_v[pl.ds(off, 16)]
                d = dst_v[pl.ds(off, 16)]
                t = typ_v[pl.ds(off, 16)]
                flat2d[j, pl.ds(u * 16, 16)] = s * R + t + tbl_off
                dst2d[j, pl.ds(u * 16, 16)] = d
                return 0
            lax.fori_loop(0, 8, _grp, 0)
            return 0
        lax.fori_loop(0, C1_ROWS, _idx, 0)

        gds = [pltpu.async_copy(xw_hbm.at[flat2d.at[j]],
                                rows_v.at[pl.ds(j * 128, 128)], sem)
               for j in range(C1_ROWS)]
        sds = []
        for j in range(C1_ROWS):
            gds[j].wait()

            def _scale(g, _):
                sv = scl_v[pl.ds(g * 16, 16)]
                for u in range(16):
                    e = g * 16 + u
                    s = sv[u]
                    rows_v[e, pl.ds(0, 16)] = rows_v[e, pl.ds(0, 16)] * s
                    rows_v[e, pl.ds(16, 16)] = rows_v[e, pl.ds(16, 16)] * s
                return 0
            lax.fori_loop(j * 8, (j + 1) * 8, _scale, 0)
            sds.append(pltpu.async_copy(rows_v.at[pl.ds(j * 128, 128)],
                                        acc.at[dst2d.at[j]], sem2, add=True))
        for d_ in sds:
            d_.wait()
        return 0

    n_mine = C1_CHUNKS // NTILE + jnp.where(sid < (C1_CHUNKS % NTILE), 1, 0)
    lax.fori_loop(0, n_mine, _chunk, 0)
    plsc.subcore_barrier()

    def _wout(nrows):
        for off in range(0, 3200, C1):
            ln = min(C1, nrows - off)
            if ln <= 0:
                break
            pltpu.sync_copy(acc.at[pl.ds(sid * NROW + off, ln)],
                            rows_v.at[pl.ds(0, ln)])
            pltpu.sync_copy(rows_v.at[pl.ds(0, ln)],
                            h_hbm.at[cid, pl.ds(sid * NROW + off, ln)])

    @pl.when(sid < NTILE - 1)
    def _():
        _wout(NROW)

    @pl.when(sid == NTILE - 1)
    def _():
        _wout(NROW_LAST)


@functools.partial(
    pl.kernel,
    out_type=jax.ShapeDtypeStruct((NSC, N, HID // 2), jnp.float32),
    mesh=_SC_MESH,
    compiler_params=_SC_PARAMS,
    scratch_types=[
        pltpu.VMEM((C1,), jnp.int32),
        pltpu.VMEM((C1,), jnp.int32),
        pltpu.VMEM((C1,), jnp.int32),
        pltpu.VMEM((C1,), jnp.float32),
        pltpu.VMEM((C1_ROWS, 128), jnp.int32),
        pltpu.VMEM((C1_ROWS, 128), jnp.int32),
        pltpu.VMEM((C1, HID // 2), jnp.float32),
        pltpu.VMEM_SHARED((N, HID // 2), jnp.float32),
        pltpu.SemaphoreType.DMA,
        pltpu.SemaphoreType.DMA,
    ],
)
def _sc_msg1(src_hbm, dst_hbm, typ_hbm, scl_hbm, xw_hbm, xr_hbm, h_hbm,
             src_v, dst_v, typ_v, scl_v, flat2d, dst2d, rows_v, acc, sem,
             sem2):
    _msg1_body(src_hbm, dst_hbm, typ_hbm, scl_hbm, xw_hbm, xr_hbm, h_hbm,
               src_v, dst_v, typ_v, scl_v, flat2d, dst2d, rows_v, acc, sem,
               sem2)


C2 = 3200
C2_ROWS = C2 // 128
C2_CHUNKS = E // NSC // C2


def _msg2_body(src_hbm, dst_hbm, typ_hbm, scl_hbm, hw_hbm, hr_hbm, p_hbm,
               src_v, dst_v, typ_v, scl_v, flat2d, dst2d, rows_v, acc, sem,
               sem2):
    cid = lax.axis_index("c")
    sid = lax.axis_index("s")

    @pl.when(cid == 0)
    def _():
        @pl.when(sid < NTILE - 1)
        def _():
            pltpu.sync_copy(hr_hbm.at[pl.ds(sid * NROW, NROW)],
                            rows_v.at[pl.ds(0, NROW)])
            pltpu.sync_copy(rows_v.at[pl.ds(0, NROW)],
                            acc.at[pl.ds(sid * NROW, NROW)])

        @pl.when(sid == NTILE - 1)
        def _():
            pltpu.sync_copy(hr_hbm.at[pl.ds(15 * NROW, NROW_LAST)],
                            rows_v.at[pl.ds(0, NROW_LAST)])
            pltpu.sync_copy(rows_v.at[pl.ds(0, NROW_LAST)],
                            acc.at[pl.ds(15 * NROW, NROW_LAST)])

    @pl.when(cid == 1)
    def _():
        def _z(i, _):
            rows_v[i, pl.ds(0, 16)] = jnp.zeros((16,), jnp.float32)
            return 0
        lax.fori_loop(0, NROW, _z, 0)

        @pl.when(sid < NTILE - 1)
        def _():
            pltpu.sync_copy(rows_v.at[pl.ds(0, NROW)],
                            acc.at[pl.ds(sid * NROW, NROW)])

        @pl.when(sid == NTILE - 1)
        def _():
            pltpu.sync_copy(rows_v.at[pl.ds(0, NROW_LAST)],
                            acc.at[pl.ds(15 * NROW, NROW_LAST)])
    plsc.subcore_barrier()

    def _chunk(k, _):
        base = (cid * (E // NSC)) + (sid + k * NTILE) * C2
        stg = [pltpu.async_copy(src_hbm.at[pl.ds(base, C2)], src_v, sem),
               pltpu.async_copy(dst_hbm.at[pl.ds(base, C2)], dst_v, sem),
               pltpu.async_copy(typ_hbm.at[pl.ds(base, C2)], typ_v, sem),
               pltpu.async_copy(scl_hbm.at[pl.ds(base, C2)], scl_v, sem)]
        for d_ in stg:
            d_.wait()

        def _idx(j, _):
            def _grp(u, _):
                off = j * 128 + u * 16
                s = src_v[pl.ds(off, 16)]
                d = dst_v[pl.ds(off, 16)]
                t = typ_v[pl.ds(off, 16)]
                flat2d[j, pl.ds(u * 16, 16)] = s * R + t
                dst2d[j, pl.ds(u * 16, 16)] = d
                return 0
            lax.fori_loop(0, 8, _grp, 0)
            return 0
        lax.fori_loop(0, C2_ROWS, _idx, 0)

        gds = [pltpu.async_copy(hw_hbm.at[flat2d.at[j]],
                                rows_v.at[pl.ds(j * 128, 128)], sem)
               for j in range(C2_ROWS)]
        sds = []
        for j in range(C2_ROWS):
            gds[j].wait()

            def _scale(g, _):
                sv = scl_v[pl.ds(g * 16, 16)]
                for u in range(16):
                    e = g * 16 + u
                    rows_v[e, pl.ds(0, 16)] = rows_v[e, pl.ds(0, 16)] * sv[u]
                return 0
            lax.fori_loop(j * 8, (j + 1) * 8, _scale, 0)
            sds.append(pltpu.async_copy(rows_v.at[pl.ds(j * 128, 128)],
                                        acc.at[dst2d.at[j]], sem2, add=True))
        for d_ in sds:
            d_.wait()
        return 0

    n_mine = C2_CHUNKS // NTILE + jnp.where(sid < (C2_CHUNKS % NTILE), 1, 0)
    lax.fori_loop(0, n_mine, _chunk, 0)
    plsc.subcore_barrier()

    @pl.when(sid < NTILE - 1)
    def _():
        pltpu.sync_copy(acc.at[pl.ds(sid * NROW, NROW)],
                        rows_v.at[pl.ds(0, NROW)])
        pltpu.sync_copy(rows_v.at[pl.ds(0, NROW)],
                        p_hbm.at[cid, pl.ds(sid * NROW, NROW)])

    @pl.when(sid == NTILE - 1)
    def _():
        pltpu.sync_copy(acc.at[pl.ds(15 * NROW, NROW_LAST)],
                        rows_v.at[pl.ds(0, NROW_LAST)])
        pltpu.sync_copy(rows_v.at[pl.ds(0, NROW_LAST)],
                        p_hbm.at[cid, pl.ds(15 * NROW, NROW_LAST)])


@functools.partial(
    pl.kernel,
    out_type=jax.ShapeDtypeStruct((NSC, N, LABELS), jnp.float32),
    mesh=_SC_MESH,
    compiler_params=_SC_PARAMS,
    scratch_types=[
        pltpu.VMEM((C2,), jnp.int32),
        pltpu.VMEM((C2,), jnp.int32),
        pltpu.VMEM((C2,), jnp.int32),
        pltpu.VMEM((C2,), jnp.float32),
        pltpu.VMEM((C2_ROWS, 128), jnp.int32),
        pltpu.VMEM((C2_ROWS, 128), jnp.int32),
        pltpu.VMEM((C2, LABELS), jnp.float32),
        pltpu.VMEM_SHARED((N, LABELS), jnp.float32),
        pltpu.SemaphoreType.DMA,
        pltpu.SemaphoreType.DMA,
    ],
)
def _sc_msg2(src_hbm, dst_hbm, typ_hbm, scl_hbm, hw_hbm, hr_hbm, p_hbm,
             src_v, dst_v, typ_v, scl_v, flat2d, dst2d, rows_v, acc, sem,
             sem2):
    _msg2_body(src_hbm, dst_hbm, typ_hbm, scl_hbm, hw_hbm, hr_hbm, p_hbm,
               src_v, dst_v, typ_v, scl_v, flat2d, dst2d, rows_v, acc, sem,
               sem2)


D1_BLK = 2000


def _dense1_body(emb_ref, w1t_ref, b1_ref, w2t_ref, b2_ref, wr1_ref,
                 root_ref, bias_ref, xw_ref, xr_ref):
    e = emb_ref[...]
    x = jax.nn.sigmoid(
        jnp.dot(e, w1t_ref[...], preferred_element_type=jnp.float32)
        + b1_ref[...])
    x = jax.nn.sigmoid(
        jnp.dot(x, w2t_ref[...], preferred_element_type=jnp.float32)
        + b2_ref[...])
    xw_ref[...] = jnp.dot(
        x, wr1_ref[0], preferred_element_type=jnp.float32).reshape(
            D1_BLK * R * (HID // 2) // 128, 128)
    xr_ref[0] = (jnp.dot(x, root_ref[0], preferred_element_type=jnp.float32)
                 + bias_ref[0])


def _dense1(emb, W1t, b1, W2t, b2, Wr1ab, rootab, biasab):
    grid = (N // D1_BLK, NSC)
    return pl.pallas_call(
        _dense1_body,
        grid=grid,
        in_specs=[
            pl.BlockSpec((D1_BLK, EMB), lambda i, h: (i, 0)),
            pl.BlockSpec((EMB, OUT_F), lambda i, h: (0, 0)),
            pl.BlockSpec((1, OUT_F), lambda i, h: (0, 0)),
            pl.BlockSpec((OUT_F, EMB), lambda i, h: (0, 0)),
            pl.BlockSpec((1, EMB), lambda i, h: (0, 0)),
            pl.BlockSpec((1, EMB, R * HID // 2), lambda i, h: (h, 0, 0)),
            pl.BlockSpec((1, EMB, HID // 2), lambda i, h: (h, 0, 0)),
            pl.BlockSpec((1, 1, HID // 2), lambda i, h: (h, 0, 0)),
        ],
        out_specs=[
            pl.BlockSpec((D1_BLK * R * (HID // 2) // 128, 128),
                         lambda i, h: (h * (N // D1_BLK) + i, 0)),
            pl.BlockSpec((1, D1_BLK, HID // 2), lambda i, h: (h, i, 0)),
        ],
        out_shape=[
            jax.ShapeDtypeStruct((NSC * NR * (HID // 2) // 128, 128),
                                 jnp.float32),
            jax.ShapeDtypeStruct((NSC, N, HID // 2), jnp.float32),
        ],
    )(emb, W1t, b1, W2t, b2, Wr1ab, rootab, biasab)


D2_BLK = 2000


def _dense2_body(ha_ref, hb_ref, wr2a_ref, wr2b_ref, root2a_ref, root2b_ref,
                 bias2_ref, hw_ref, hr_ref):
    ha = jax.nn.relu(ha_ref[0])
    hb = jax.nn.relu(hb_ref[0])
    hw_ref[...] = (
        jnp.dot(ha, wr2a_ref[...], preferred_element_type=jnp.float32)
        + jnp.dot(hb, wr2b_ref[...], preferred_element_type=jnp.float32))
    hr_ref[...] = (
        jnp.dot(ha, root2a_ref[...], preferred_element_type=jnp.float32)
        + jnp.dot(hb, root2b_ref[...], preferred_element_type=jnp.float32)
        + bias2_ref[...])


def _dense2(h, Wr2a, Wr2b, root2a, root2b, bias2):
    grid = (N // D2_BLK,)
    return pl.pallas_call(
        _dense2_body,
        grid=grid,
        in_specs=[
            pl.BlockSpec((1, D2_BLK, HID // 2), lambda i: (0, i, 0)),
            pl.BlockSpec((1, D2_BLK, HID // 2), lambda i: (1, i, 0)),
            pl.BlockSpec((HID // 2, R * LABELS), lambda i: (0, 0)),
            pl.BlockSpec((HID // 2, R * LABELS), lambda i: (0, 0)),
            pl.BlockSpec((HID // 2, LABELS), lambda i: (0, 0)),
            pl.BlockSpec((HID // 2, LABELS), lambda i: (0, 0)),
            pl.BlockSpec((1, LABELS), lambda i: (0, 0)),
        ],
        out_specs=[
            pl.BlockSpec((D2_BLK, R * LABELS), lambda i: (i, 0)),
            pl.BlockSpec((D2_BLK, LABELS), lambda i: (i, 0)),
        ],
        out_shape=[
            jax.ShapeDtypeStruct((N, R * LABELS), jnp.float32),
            jax.ShapeDtypeStruct((N, LABELS), jnp.float32),
        ],
    )(h, h, Wr2a, Wr2b, root2a, root2b, bias2)


def _final_body(p_ref, out_ref):
    out_ref[...] = jax.nn.sigmoid(p_ref[0] + p_ref[1])


def _final(p):
    grid = (N // D2_BLK,)
    return pl.pallas_call(
        _final_body,
        grid=grid,
        in_specs=[pl.BlockSpec((NSC, D2_BLK, LABELS), lambda i: (0, i, 0))],
        out_specs=pl.BlockSpec((D2_BLK, LABELS), lambda i: (i, 0)),
        out_shape=jax.ShapeDtypeStruct((N, LABELS), jnp.float32),
    )(p)


def kernel(emb, W1, b1, W2, b2, Wr1, root1, bias1, Wr2, root2, bias2,
           edge_index, edge_type):
    src = edge_index[0]
    dst = edge_index[1]

    edge_scale = _sc_counts(dst, edge_type)

    W1t = W1.T
    W2t = W2.T
    Wr1f = Wr1.transpose(1, 0, 2)
    Wr1ab = jnp.stack([Wr1f[:, :, :HID // 2].reshape(EMB, R * HID // 2),
                       Wr1f[:, :, HID // 2:].reshape(EMB, R * HID // 2)])
    rootab = jnp.stack([root1[:, :HID // 2], root1[:, HID // 2:]])
    biasab = jnp.stack([bias1[:HID // 2].reshape(1, -1),
                        bias1[HID // 2:].reshape(1, -1)])

    xw, xr = _dense1(emb, W1t, b1.reshape(1, -1), W2t, b2.reshape(1, -1),
                     Wr1ab, rootab, biasab)

    h_raw = _sc_msg1(src, dst, edge_type, edge_scale,
                     xw.reshape(NSC * NR, HID // 2), xr)

    Wr2f = Wr2.transpose(1, 0, 2)
    Wr2a = Wr2f[:HID // 2].reshape(HID // 2, R * LABELS)
    Wr2b = Wr2f[HID // 2:].reshape(HID // 2, R * LABELS)
    root2a = root2[:HID // 2]
    root2b = root2[HID // 2:]

    hw, hr = _dense2(h_raw, Wr2a, Wr2b, root2a, root2b, bias2.reshape(1, -1))

    p = _sc_msg2(src, dst, edge_type, edge_scale, hw.reshape(NR, LABELS), hr)

    return _final(p)


# speedup vs baseline: 27.2351x; 1.0354x over previous
import functools
import jax
import jax.numpy as jnp
from jax import lax
from jax.experimental import pallas as pl
from jax.experimental.pallas import tpu as pltpu
from jax.experimental.pallas import tpu_sc as plsc

N = 50000
E = 800000
R = 8
EMB = 64
HID = 64
LABELS = 16
OUT_F = 112

ROW_BLK = 400

NSC = 2
NTILE = 16
NR = N * R
SLICE_W = 25024
NR_PAD = NTILE * SLICE_W
NROW = 3128
NROW_LAST = N - 15 * NROW

_SC_MESH = plsc.VectorSubcoreMesh(
    core_axis_name="c", subcore_axis_name="s",
    num_cores=NSC, num_subcores=NTILE)
_SC_PARAMS = pltpu.CompilerParams(use_tc_tiling_on_sc=False)


C_CNT = 3200
CNT_ROWS = C_CNT // 128
CNT_CHUNKS = E // C_CNT
SCL_CHUNKS = E // NSC // C_CNT


def _counts_body(edge_hbm, typ_hbm, scl_hbm, dst_v, typ_v, comp_v, ones_v,
                 sweep_v, invv, table, sem, sem2):
    cid = lax.axis_index("c")
    sid = lax.axis_index("s")

    def _fill(i, _):
        ones_v[pl.ds(i * 16, 16)] = jnp.full((16,), 1.0, jnp.float32)
        return 0
    lax.fori_loop(0, 128 // 16, _fill, 0)

    def _zero(i, _):
        sweep_v[pl.ds(i * 16, 16)] = jnp.zeros((16,), jnp.float32)
        return 0
    lax.fori_loop(0, SLICE_W // 16, _zero, 0)

    pltpu.sync_copy(sweep_v, table.at[pl.ds(sid * SLICE_W, SLICE_W)])
    plsc.subcore_barrier()

    def _chunk(k, _):
        base = (sid + k * NTILE) * C_CNT
        stg = [pltpu.async_copy(edge_hbm.at[1, pl.ds(base, C_CNT)], dst_v, sem),
               pltpu.async_copy(typ_hbm.at[pl.ds(base, C_CNT)], typ_v, sem)]
        for d_ in stg:
            d_.wait()

        def _comp(j, _):
            def _grp(u, _):
                off = j * 128 + u * 16
                d = dst_v[pl.ds(off, 16)]
                t = typ_v[pl.ds(off, 16)]
                comp_v[j, pl.ds(u * 16, 16)] = d * R + t
                return 0
            lax.fori_loop(0, 8, _grp, 0)
            return 0
        lax.fori_loop(0, CNT_ROWS, _comp, 0)

        scds = [pltpu.async_copy(ones_v, table.at[comp_v.at[j]], sem2,
                                 add=True)
                for j in range(CNT_ROWS)]
        for d_ in scds:
            d_.wait()
        return 0

    n_mine = CNT_CHUNKS // NTILE + jnp.where(sid < (CNT_CHUNKS % NTILE), 1, 0)
    lax.fori_loop(0, n_mine, _chunk, 0)
    plsc.subcore_barrier()

    pltpu.sync_copy(table.at[pl.ds(sid * SLICE_W, SLICE_W)], sweep_v)

    def _inv(i, _):
        c = sweep_v[pl.ds(i * 16, 16)]
        sweep_v[pl.ds(i * 16, 16)] = jnp.where(
            c > 0.0, 1.0 / jnp.maximum(c, 1.0), 0.0)
        return 0
    lax.fori_loop(0, SLICE_W // 16, _inv, 0)
    pltpu.sync_copy(sweep_v, table.at[pl.ds(sid * SLICE_W, SLICE_W)])
    plsc.subcore_barrier()

    def _schunk(k, _):
        base = cid * (E // NSC) + (sid + k * NTILE) * C_CNT
        pltpu.sync_copy(edge_hbm.at[1, pl.ds(base, C_CNT)], dst_v)
        pltpu.sync_copy(typ_hbm.at[pl.ds(base, C_CNT)], typ_v)

        def _comp(j, _):
            def _grp(u, _):
                off = j * 128 + u * 16
                d = dst_v[pl.ds(off, 16)]
                t = typ_v[pl.ds(off, 16)]
                comp_v[j, pl.ds(u * 16, 16)] = d * R + t
                return 0
            lax.fori_loop(0, 8, _grp, 0)
            return 0
        lax.fori_loop(0, CNT_ROWS, _comp, 0)

        gds = [pltpu.async_copy(table.at[comp_v.at[j]],
                                invv.at[pl.ds(j * 128, 128)], sem2)
               for j in range(CNT_ROWS)]
        for d_ in gds:
            d_.wait()
        pltpu.sync_copy(invv, scl_hbm.at[pl.ds(base, C_CNT)])
        return 0

    n_mine2 = SCL_CHUNKS // NTILE + jnp.where(sid < (SCL_CHUNKS % NTILE), 1, 0)
    lax.fori_loop(0, n_mine2, _schunk, 0)


@functools.partial(
    pl.kernel,
    out_type=jax.ShapeDtypeStruct((E,), jnp.float32),
    mesh=_SC_MESH,
    compiler_params=_SC_PARAMS,
    scratch_types=[
        pltpu.VMEM((C_CNT,), jnp.int32),
        pltpu.VMEM((C_CNT,), jnp.int32),
        pltpu.VMEM((CNT_ROWS, 128), jnp.int32),
        pltpu.VMEM((128,), jnp.float32),
        pltpu.VMEM((SLICE_W,), jnp.float32),
        pltpu.VMEM((C_CNT,), jnp.float32),
        pltpu.VMEM_SHARED((NR_PAD,), jnp.float32),
        pltpu.SemaphoreType.DMA,
        pltpu.SemaphoreType.DMA,
    ],
)
def _sc_counts(edge_hbm, typ_hbm, scl_hbm, dst_v, typ_v, comp_v, ones_v,
               sweep_v, invv, table, sem, sem2):
    _counts_body(edge_hbm, typ_hbm, scl_hbm, dst_v, typ_v, comp_v, ones_v,
                 sweep_v, invv, table, sem, sem2)


C1 = 640
C1_ROWS = C1 // 128
C1_CHUNKS = E // C1


def _msg1_body(edge_hbm, typ_hbm, scl_hbm, xw_hbm, xr_hbm, h_hbm,
               src_v, dst_v, typ_v, scl_v, flat2d, dst2d, rows_v, acc, sem,
               sem2):
    cid = lax.axis_index("c")
    sid = lax.axis_index("s")
    tbl_off = cid * NR

    def _init(nrows):
        for off in range(0, 3200, C1):
            ln = min(C1, nrows - off)
            if ln <= 0:
                break
            pltpu.sync_copy(xr_hbm.at[cid, pl.ds(sid * NROW + off, ln)],
                            rows_v.at[pl.ds(0, ln)])
            pltpu.sync_copy(rows_v.at[pl.ds(0, ln)],
                            acc.at[pl.ds(sid * NROW + off, ln)])

    @pl.when(sid < NTILE - 1)
    def _():
        _init(NROW)

    @pl.when(sid == NTILE - 1)
    def _():
        _init(NROW_LAST)
    plsc.subcore_barrier()

    def _chunk(k, _):
        base = (sid + k * NTILE) * C1
        stg = [pltpu.async_copy(edge_hbm.at[0, pl.ds(base, C1)], src_v, sem),
               pltpu.async_copy(edge_hbm.at[1, pl.ds(base, C1)], dst_v, sem),
               pltpu.async_copy(typ_hbm.at[pl.ds(base, C1)], typ_v, sem),
               pltpu.async_copy(scl_hbm.at[pl.ds(base, C1)], scl_v, sem)]
        for d_ in stg:
            d_.wait()

        def _idx(j, _):
            def _grp(u, _):
                off = j * 128 + u * 16
                s = src_v[pl.ds(off, 16)]
                d = dst_v[pl.ds(off, 16)]
                t = typ_v[pl.ds(off, 16)]
                flat2d[j, pl.ds(u * 16, 16)] = s * R + t + tbl_off
                dst2d[j, pl.ds(u * 16, 16)] = d
                return 0
            lax.fori_loop(0, 8, _grp, 0)
            return 0
        lax.fori_loop(0, C1_ROWS, _idx, 0)

        gds = [pltpu.async_copy(xw_hbm.at[flat2d.at[j]],
                                rows_v.at[pl.ds(j * 128, 128)], sem)
               for j in range(C1_ROWS)]
        sds = []
        for j in range(C1_ROWS):
            gds[j].wait()

            def _scale(g, _):
                sv = scl_v[pl.ds(g * 16, 16)]
                for u in range(16):
                    e = g * 16 + u
                    s = sv[u]
                    rows_v[e, pl.ds(0, 16)] = rows_v[e, pl.ds(0, 16)] * s
                    rows_v[e, pl.ds(16, 16)] = rows_v[e, pl.ds(16, 16)] * s
                return 0
            lax.fori_loop(j * 8, (j + 1) * 8, _scale, 0)
            sds.append(pltpu.async_copy(rows_v.at[pl.ds(j * 128, 128)],
                                        acc.at[dst2d.at[j]], sem2, add=True))
        for d_ in sds:
            d_.wait()
        return 0

    n_mine = C1_CHUNKS // NTILE + jnp.where(sid < (C1_CHUNKS % NTILE), 1, 0)
    lax.fori_loop(0, n_mine, _chunk, 0)
    plsc.subcore_barrier()

    def _wout(nrows):
        for off in range(0, 3200, C1):
            ln = min(C1, nrows - off)
            if ln <= 0:
                break
            pltpu.sync_copy(acc.at[pl.ds(sid * NROW + off, ln)],
                            rows_v.at[pl.ds(0, ln)])
            pltpu.sync_copy(rows_v.at[pl.ds(0, ln)],
                            h_hbm.at[cid, pl.ds(sid * NROW + off, ln)])

    @pl.when(sid < NTILE - 1)
    def _():
        _wout(NROW)

    @pl.when(sid == NTILE - 1)
    def _():
        _wout(NROW_LAST)


@functools.partial(
    pl.kernel,
    out_type=jax.ShapeDtypeStruct((NSC, N, HID // 2), jnp.float32),
    mesh=_SC_MESH,
    compiler_params=_SC_PARAMS,
    scratch_types=[
        pltpu.VMEM((C1,), jnp.int32),
        pltpu.VMEM((C1,), jnp.int32),
        pltpu.VMEM((C1,), jnp.int32),
        pltpu.VMEM((C1,), jnp.float32),
        pltpu.VMEM((C1_ROWS, 128), jnp.int32),
        pltpu.VMEM((C1_ROWS, 128), jnp.int32),
        pltpu.VMEM((C1, HID // 2), jnp.float32),
        pltpu.VMEM_SHARED((N, HID // 2), jnp.float32),
        pltpu.SemaphoreType.DMA,
        pltpu.SemaphoreType.DMA,
    ],
)
def _sc_msg1(edge_hbm, typ_hbm, scl_hbm, xw_hbm, xr_hbm, h_hbm,
             src_v, dst_v, typ_v, scl_v, flat2d, dst2d, rows_v, acc, sem,
             sem2):
    _msg1_body(edge_hbm, typ_hbm, scl_hbm, xw_hbm, xr_hbm, h_hbm,
               src_v, dst_v, typ_v, scl_v, flat2d, dst2d, rows_v, acc, sem,
               sem2)


C2 = 3200
C2_ROWS = C2 // 128
C2_CHUNKS = E // NSC // C2


def _msg2_body(edge_hbm, typ_hbm, scl_hbm, hw_hbm, hr_hbm, p_hbm,
               src_v, dst_v, typ_v, scl_v, flat2d, dst2d, rows_v, acc, sem,
               sem2):
    cid = lax.axis_index("c")
    sid = lax.axis_index("s")

    @pl.when(cid == 0)
    def _():
        @pl.when(sid < NTILE - 1)
        def _():
            pltpu.sync_copy(hr_hbm.at[pl.ds(sid * NROW, NROW)],
                            rows_v.at[pl.ds(0, NROW)])
            pltpu.sync_copy(rows_v.at[pl.ds(0, NROW)],
                            acc.at[pl.ds(sid * NROW, NROW)])

        @pl.when(sid == NTILE - 1)
        def _():
            pltpu.sync_copy(hr_hbm.at[pl.ds(15 * NROW, NROW_LAST)],
                            rows_v.at[pl.ds(0, NROW_LAST)])
            pltpu.sync_copy(rows_v.at[pl.ds(0, NROW_LAST)],
                            acc.at[pl.ds(15 * NROW, NROW_LAST)])

    @pl.when(cid == 1)
    def _():
        def _z(i, _):
            rows_v[i, pl.ds(0, 16)] = jnp.zeros((16,), jnp.float32)
            return 0
        lax.fori_loop(0, NROW, _z, 0)

        @pl.when(sid < NTILE - 1)
        def _():
            pltpu.sync_copy(rows_v.at[pl.ds(0, NROW)],
                            acc.at[pl.ds(sid * NROW, NROW)])

        @pl.when(sid == NTILE - 1)
        def _():
            pltpu.sync_copy(rows_v.at[pl.ds(0, NROW_LAST)],
                            acc.at[pl.ds(15 * NROW, NROW_LAST)])
    plsc.subcore_barrier()

    def _chunk(k, _):
        base = (cid * (E // NSC)) + (sid + k * NTILE) * C2
        stg = [pltpu.async_copy(edge_hbm.at[0, pl.ds(base, C2)], src_v, sem),
               pltpu.async_copy(edge_hbm.at[1, pl.ds(base, C2)], dst_v, sem),
               pltpu.async_copy(typ_hbm.at[pl.ds(base, C2)], typ_v, sem),
               pltpu.async_copy(scl_hbm.at[pl.ds(base, C2)], scl_v, sem)]
        for d_ in stg:
            d_.wait()

        def _idx(j, _):
            def _grp(u, _):
                off = j * 128 + u * 16
                s = src_v[pl.ds(off, 16)]
                d = dst_v[pl.ds(off, 16)]
                t = typ_v[pl.ds(off, 16)]
                flat2d[j, pl.ds(u * 16, 16)] = s * R + t
                dst2d[j, pl.ds(u * 16, 16)] = d
                return 0
            lax.fori_loop(0, 8, _grp, 0)
            return 0
        lax.fori_loop(0, C2_ROWS, _idx, 0)

        gds = [pltpu.async_copy(hw_hbm.at[flat2d.at[j]],
                                rows_v.at[pl.ds(j * 128, 128)], sem)
               for j in range(C2_ROWS)]
        sds = []
        for j in range(C2_ROWS):
            gds[j].wait()

            def _scale(g, _):
                sv = scl_v[pl.ds(g * 16, 16)]
                for u in range(16):
                    e = g * 16 + u
                    rows_v[e, pl.ds(0, 16)] = rows_v[e, pl.ds(0, 16)] * sv[u]
                return 0
            lax.fori_loop(j * 8, (j + 1) * 8, _scale, 0)
            sds.append(pltpu.async_copy(rows_v.at[pl.ds(j * 128, 128)],
                                        acc.at[dst2d.at[j]], sem2, add=True))
        for d_ in sds:
            d_.wait()
        return 0

    n_mine = C2_CHUNKS // NTILE + jnp.where(sid < (C2_CHUNKS % NTILE), 1, 0)
    lax.fori_loop(0, n_mine, _chunk, 0)
    plsc.subcore_barrier()

    @pl.when(sid < NTILE - 1)
    def _():
        pltpu.sync_copy(acc.at[pl.ds(sid * NROW, NROW)],
                        rows_v.at[pl.ds(0, NROW)])
        pltpu.sync_copy(rows_v.at[pl.ds(0, NROW)],
                        p_hbm.at[cid, pl.ds(sid * NROW, NROW)])

    @pl.when(sid == NTILE - 1)
    def _():
        pltpu.sync_copy(acc.at[pl.ds(15 * NROW, NROW_LAST)],
                        rows_v.at[pl.ds(0, NROW_LAST)])
        pltpu.sync_copy(rows_v.at[pl.ds(0, NROW_LAST)],
                        p_hbm.at[cid, pl.ds(15 * NROW, NROW_LAST)])


@functools.partial(
    pl.kernel,
    out_type=jax.ShapeDtypeStruct((NSC, N, LABELS), jnp.float32),
    mesh=_SC_MESH,
    compiler_params=_SC_PARAMS,
    scratch_types=[
        pltpu.VMEM((C2,), jnp.int32),
        pltpu.VMEM((C2,), jnp.int32),
        pltpu.VMEM((C2,), jnp.int32),
        pltpu.VMEM((C2,), jnp.float32),
        pltpu.VMEM((C2_ROWS, 128), jnp.int32),
        pltpu.VMEM((C2_ROWS, 128), jnp.int32),
        pltpu.VMEM((C2, LABELS), jnp.float32),
        pltpu.VMEM_SHARED((N, LABELS), jnp.float32),
        pltpu.SemaphoreType.DMA,
        pltpu.SemaphoreType.DMA,
    ],
)
def _sc_msg2(edge_hbm, typ_hbm, scl_hbm, hw_hbm, hr_hbm, p_hbm,
             src_v, dst_v, typ_v, scl_v, flat2d, dst2d, rows_v, acc, sem,
             sem2):
    _msg2_body(edge_hbm, typ_hbm, scl_hbm, hw_hbm, hr_hbm, p_hbm,
               src_v, dst_v, typ_v, scl_v, flat2d, dst2d, rows_v, acc, sem,
               sem2)


D1_BLK = 2000


def _dense1_body(emb_ref, w1t_ref, b1_ref, w2t_ref, b2_ref, wr1_ref,
                 root_ref, bias_ref, xw_ref, xr_ref):
    e = emb_ref[...]
    x = jax.nn.sigmoid(
        jnp.dot(e, w1t_ref[...], preferred_element_type=jnp.float32)
        + b1_ref[...])
    x = jax.nn.sigmoid(
        jnp.dot(x, w2t_ref[...], preferred_element_type=jnp.float32)
        + b2_ref[...])
    xw_ref[...] = jnp.dot(
        x, wr1_ref[0], preferred_element_type=jnp.float32).reshape(
            D1_BLK * R * (HID // 2) // 128, 128)
    xr_ref[0] = (jnp.dot(x, root_ref[0], preferred_element_type=jnp.float32)
                 + bias_ref[0])


def _dense1(emb, W1t, b1, W2t, b2, Wr1ab, rootab, biasab):
    grid = (N // D1_BLK, NSC)
    return pl.pallas_call(
        _dense1_body,
        grid=grid,
        in_specs=[
            pl.BlockSpec((D1_BLK, EMB), lambda i, h: (i, 0)),
            pl.BlockSpec((EMB, OUT_F), lambda i, h: (0, 0)),
            pl.BlockSpec((1, OUT_F), lambda i, h: (0, 0)),
            pl.BlockSpec((OUT_F, EMB), lambda i, h: (0, 0)),
            pl.BlockSpec((1, EMB), lambda i, h: (0, 0)),
            pl.BlockSpec((1, EMB, R * HID // 2), lambda i, h: (h, 0, 0)),
            pl.BlockSpec((1, EMB, HID // 2), lambda i, h: (h, 0, 0)),
            pl.BlockSpec((1, 1, HID // 2), lambda i, h: (h, 0, 0)),
        ],
        out_specs=[
            pl.BlockSpec((D1_BLK * R * (HID // 2) // 128, 128),
                         lambda i, h: (h * (N // D1_BLK) + i, 0)),
            pl.BlockSpec((1, D1_BLK, HID // 2), lambda i, h: (h, i, 0)),
        ],
        out_shape=[
            jax.ShapeDtypeStruct((NSC * NR * (HID // 2) // 128, 128),
                                 jnp.float32),
            jax.ShapeDtypeStruct((NSC, N, HID // 2), jnp.float32),
        ],
    )(emb, W1t, b1, W2t, b2, Wr1ab, rootab, biasab)


D2_BLK = 2000


def _dense2_body(ha_ref, hb_ref, wr2a_ref, wr2b_ref, root2a_ref, root2b_ref,
                 bias2_ref, hw_ref, hr_ref):
    ha = jax.nn.relu(ha_ref[0])
    hb = jax.nn.relu(hb_ref[0])
    hw_ref[...] = (
        jnp.dot(ha, wr2a_ref[...], preferred_element_type=jnp.float32)
        + jnp.dot(hb, wr2b_ref[...], preferred_element_type=jnp.float32))
    hr_ref[...] = (
        jnp.dot(ha, root2a_ref[...], preferred_element_type=jnp.float32)
        + jnp.dot(hb, root2b_ref[...], preferred_element_type=jnp.float32)
        + bias2_ref[...])


def _dense2(h, Wr2a, Wr2b, root2a, root2b, bias2):
    grid = (N // D2_BLK,)
    return pl.pallas_call(
        _dense2_body,
        grid=grid,
        in_specs=[
            pl.BlockSpec((1, D2_BLK, HID // 2), lambda i: (0, i, 0)),
            pl.BlockSpec((1, D2_BLK, HID // 2), lambda i: (1, i, 0)),
            pl.BlockSpec((HID // 2, R * LABELS), lambda i: (0, 0)),
            pl.BlockSpec((HID // 2, R * LABELS), lambda i: (0, 0)),
            pl.BlockSpec((HID // 2, LABELS), lambda i: (0, 0)),
            pl.BlockSpec((HID // 2, LABELS), lambda i: (0, 0)),
            pl.BlockSpec((1, LABELS), lambda i: (0, 0)),
        ],
        out_specs=[
            pl.BlockSpec((D2_BLK, R * LABELS), lambda i: (i, 0)),
            pl.BlockSpec((D2_BLK, LABELS), lambda i: (i, 0)),
        ],
        out_shape=[
            jax.ShapeDtypeStruct((N, R * LABELS), jnp.float32),
            jax.ShapeDtypeStruct((N, LABELS), jnp.float32),
        ],
    )(h, h, Wr2a, Wr2b, root2a, root2b, bias2)


F_ROWS = 2000


def _sc_final_body(p_hbm, out_hbm, pa, pb, sem):
    cid = lax.axis_index("c")
    sid = lax.axis_index("s")
    wid = cid * NTILE + sid

    @pl.when(wid < N // F_ROWS)
    def _():
        base = wid * F_ROWS
        stg = [pltpu.async_copy(p_hbm.at[0, pl.ds(base, F_ROWS)], pa, sem),
               pltpu.async_copy(p_hbm.at[1, pl.ds(base, F_ROWS)], pb, sem)]
        for d_ in stg:
            d_.wait()

        def _sig(i, _):
            a = pa[i, pl.ds(0, 16)]
            b = pb[i, pl.ds(0, 16)]
            s = a + b
            pa[i, pl.ds(0, 16)] = 1.0 / (1.0 + jnp.exp(-s))
            return 0
        lax.fori_loop(0, F_ROWS, _sig, 0)
        pltpu.sync_copy(pa, out_hbm.at[pl.ds(base, F_ROWS)])


@functools.partial(
    pl.kernel,
    out_type=jax.ShapeDtypeStruct((N, LABELS), jnp.float32),
    mesh=_SC_MESH,
    compiler_params=_SC_PARAMS,
    scratch_types=[
        pltpu.VMEM((F_ROWS, LABELS), jnp.float32),
        pltpu.VMEM((F_ROWS, LABELS), jnp.float32),
        pltpu.SemaphoreType.DMA,
    ],
)
def _sc_final(p_hbm, out_hbm, pa, pb, sem):
    _sc_final_body(p_hbm, out_hbm, pa, pb, sem)


def kernel(emb, W1, b1, W2, b2, Wr1, root1, bias1, Wr2, root2, bias2,
           edge_index, edge_type):
    edge_scale = _sc_counts(edge_index, edge_type)

    W1t = W1.T
    W2t = W2.T
    Wr1f = Wr1.transpose(1, 0, 2)
    Wr1ab = jnp.stack([Wr1f[:, :, :HID // 2].reshape(EMB, R * HID // 2),
                       Wr1f[:, :, HID // 2:].reshape(EMB, R * HID // 2)])
    rootab = jnp.stack([root1[:, :HID // 2], root1[:, HID // 2:]])
    biasab = jnp.stack([bias1[:HID // 2].reshape(1, -1),
                        bias1[HID // 2:].reshape(1, -1)])

    xw, xr = _dense1(emb, W1t, b1.reshape(1, -1), W2t, b2.reshape(1, -1),
                     Wr1ab, rootab, biasab)

    h_raw = _sc_msg1(edge_index, edge_type, edge_scale,
                     xw.reshape(NSC * NR, HID // 2), xr)

    Wr2f = Wr2.transpose(1, 0, 2)
    Wr2a = Wr2f[:HID // 2].reshape(HID // 2, R * LABELS)
    Wr2b = Wr2f[HID // 2:].reshape(HID // 2, R * LABELS)
    root2a = root2[:HID // 2]
    root2b = root2[HID // 2:]

    hw, hr = _dense2(h_raw, Wr2a, Wr2b, root2a, root2b, bias2.reshape(1, -1))

    p = _sc_msg2(edge_index, edge_type, edge_scale, hw.reshape(NR, LABELS), hr)

    return _sc_final(p)
